# Initial kernel scaffold; baseline (speedup 1.0000x reference)
import jax
import jax.numpy as jnp
from jax.experimental import pallas as pl


def kernel(f, x, x_init, edge_index, edge_type, f_batch, x_batch, W_rel, W_self, b_gcn, Wq, Wk, Wv, Wf, Wi, a_vec, W_lin, b_lin):
    raise NotImplementedError("write your pallas kernel here")


# speedup vs baseline: 1.3291x; 1.3291x over previous
import functools
import math

import jax
import jax.numpy as jnp
from jax import lax
from jax.experimental import pallas as pl
from jax.experimental.pallas import tpu as pltpu

N = 10000
E = 320000
G = 16
R = 8
D = 128

BN = 1000


def _dot(a, b):
    return lax.dot_general(a, b, (((1,), (0,)), ((), ())),
                           precision=lax.Precision.HIGHEST,
                           preferred_element_type=jnp.float32)


def _phase1_body(x_ref, xi_ref, wrel_ref, wself_ref, bgcn_ref, wk_ref, wv_ref,
                 wia_ref, xw_ref, k_ref, v_ref, xself_ref, ctxd_ref):
    xb = x_ref[...]
    for r in range(R):
        xw_ref[r] = _dot(xb, wrel_ref[r])
    k_ref[...] = _dot(xb, wk_ref[...])
    v_ref[...] = _dot(xb, wv_ref[...])
    xself_ref[...] = _dot(xb, wself_ref[...]) + bgcn_ref[...]
    ctxd_ref[...] = _dot(xi_ref[...], wia_ref[...])


def _phase1(x, x_init, W_rel, W_self, b_gcn, Wk, Wv, wia):
    grid = (N // BN,)
    full = lambda i: (0, 0)
    out_shapes = (
        jax.ShapeDtypeStruct((R, N, D), jnp.float32),
        jax.ShapeDtypeStruct((N, D), jnp.float32),
        jax.ShapeDtypeStruct((N, D), jnp.float32),
        jax.ShapeDtypeStruct((N, D), jnp.float32),
        jax.ShapeDtypeStruct((N, 1), jnp.float32),
    )
    return pl.pallas_call(
        _phase1_body,
        grid=grid,
        in_specs=[
            pl.BlockSpec((BN, D), lambda i: (i, 0)),
            pl.BlockSpec((BN, D), lambda i: (i, 0)),
            pl.BlockSpec((R, D, D), lambda i: (0, 0, 0)),
            pl.BlockSpec((D, D), full),
            pl.BlockSpec((1, D), full),
            pl.BlockSpec((D, D), full),
            pl.BlockSpec((D, D), full),
            pl.BlockSpec((D, 1), full),
        ],
        out_specs=(
            pl.BlockSpec((R, BN, D), lambda i: (0, i, 0)),
            pl.BlockSpec((BN, D), lambda i: (i, 0)),
            pl.BlockSpec((BN, D), lambda i: (i, 0)),
            pl.BlockSpec((BN, D), lambda i: (i, 0)),
            pl.BlockSpec((BN, 1), lambda i: (i, 0)),
        ),
        out_shape=out_shapes,
    )(x, x_init, W_rel, W_self, b_gcn.reshape(1, D), Wk, Wv, wia)


def _phase2_body(agg_ref, deg_ref, xself_ref, wq_ref, h_ref, q_ref):
    degc = jnp.maximum(deg_ref[...], 1.0)
    h = agg_ref[...] / degc + xself_ref[...]
    h_ref[...] = h
    q_ref[...] = _dot(h, wq_ref[...])


def _phase2(agg, deg, xself, Wq_s):
    grid = (N // BN,)
    return pl.pallas_call(
        _phase2_body,
        grid=grid,
        in_specs=[
            pl.BlockSpec((BN, D), lambda i: (i, 0)),
            pl.BlockSpec((BN, 1), lambda i: (i, 0)),
            pl.BlockSpec((BN, D), lambda i: (i, 0)),
            pl.BlockSpec((D, D), lambda i: (0, 0)),
        ],
        out_specs=(
            pl.BlockSpec((BN, D), lambda i: (i, 0)),
            pl.BlockSpec((BN, D), lambda i: (i, 0)),
        ),
        out_shape=(
            jax.ShapeDtypeStruct((N, D), jnp.float32),
            jax.ShapeDtypeStruct((N, D), jnp.float32),
        ),
    )(agg, deg.reshape(N, 1), xself, Wq_s)


def _phase3_body(h_ref, cacc_ref, dinv_ref, wlin_ref, blin_ref, z_ref):
    z = h_ref[...] + cacc_ref[...] * dinv_ref[...]
    z_ref[...] = _dot(z, wlin_ref[...]) + blin_ref[...]


def _phase3(h, cacc, dinv, W_lin, b_lin):
    grid = (N // BN,)
    return pl.pallas_call(
        _phase3_body,
        grid=grid,
        in_specs=[
            pl.BlockSpec((BN, D), lambda i: (i, 0)),
            pl.BlockSpec((BN, D), lambda i: (i, 0)),
            pl.BlockSpec((BN, 1), lambda i: (i, 0)),
            pl.BlockSpec((D, D), lambda i: (0, 0)),
            pl.BlockSpec((1, D), lambda i: (0, 0)),
        ],
        out_specs=pl.BlockSpec((BN, D), lambda i: (i, 0)),
        out_shape=jax.ShapeDtypeStruct((N, D), jnp.float32),
    )(h, cacc, dinv.reshape(N, 1), W_lin, b_lin.reshape(1, D))


def kernel(f, x, x_init, edge_index, edge_type, f_batch, x_batch, W_rel,
           W_self, b_gcn, Wq, Wk, Wv, Wf, Wi, a_vec, W_lin, b_lin):
    src = edge_index[0]
    dst = edge_index[1]

    fw = f @ Wf
    f_g = jax.ops.segment_sum(fw, f_batch, num_segments=G)
    f_cnt = jax.ops.segment_sum(jnp.ones((G,), x.dtype), f_batch,
                                num_segments=G)
    f_g = f_g / jnp.clip(f_cnt, 1.0)[:, None]
    fdot = f_g @ a_vec

    wia = (Wi @ a_vec).reshape(D, 1)
    Wq_s = Wq * (1.0 / math.sqrt(D))

    xw, k, v, xself, ctxd = _phase1(x, x_init, W_rel, W_self, b_gcn, Wk, Wv,
                                    wia)
    ctxs = ctxd[:, 0] + fdot[x_batch]

    xw2 = xw.reshape(R * N, D)
    gidx = edge_type * N + src

    msg = jnp.take(xw2, gidx, axis=0)
    deg = jax.ops.segment_sum(jnp.ones((E,), x.dtype), dst, num_segments=N)
    agg = jax.ops.segment_sum(msg, dst, num_segments=N)

    h, q = _phase2(agg, deg, xself, Wq_s)

    score = (jnp.take(q, dst, axis=0) * jnp.take(k, src, axis=0)).sum(-1)
    score = score + jnp.take(ctxs, dst)
    score = jnp.where(score >= 0, score, 0.2 * score)

    smax = jax.ops.segment_max(score, dst, num_segments=N)
    smax = jnp.where(jnp.isfinite(smax), smax, 0.0)

    ex = jnp.exp(score - jnp.take(smax, dst))
    denom = jax.ops.segment_sum(ex, dst, num_segments=N)
    cacc = jax.ops.segment_sum(ex[:, None] * jnp.take(v, src, axis=0), dst,
                               num_segments=N)
    dinv = 1.0 / jnp.clip(denom, 1e-16)
    alpha = ex * jnp.take(dinv, dst)

    z = _phase3(h, cacc, dinv, W_lin, b_lin)
    return (z, alpha)


# speedup vs baseline: 1.4591x; 1.0978x over previous
import functools
import math

import jax
import jax.numpy as jnp
from jax import lax
from jax.experimental import pallas as pl
from jax.experimental.pallas import tpu as pltpu
from jax.experimental.pallas import tpu_sc as plsc

N = 10000
E = 320000
G = 16
R = 8
D = 128

BN = 1000

NW = 32
CH = 128
EPW = 10112
NCH = EPW // CH
EPAD = NW * EPW
NACC = 10240
TW = 144

def _mesh():
    return plsc.VectorSubcoreMesh(core_axis_name="c", subcore_axis_name="s")


def _sc_pass_a(table, gidx_p, dsts_p, zacc):
    @functools.partial(
        pl.kernel,
        mesh=_mesh(),
        compiler_params=pltpu.CompilerParams(use_tc_tiling_on_sc=False),
        out_type=jax.ShapeDtypeStruct((2, NACC, TW), jnp.float32),
        scratch_types=[
            pltpu.VMEM((CH,), jnp.int32),
            pltpu.VMEM((CH,), jnp.int32),
            pltpu.VMEM((CH, TW), jnp.float32),
            pltpu.VMEM_SHARED((NACC, TW), jnp.float32),
        ],
    )
    def k(table_h, gidx_h, dst_h, zacc_h, out_h, gi_v, di_v, rows_v, acc_sh):
        c = lax.axis_index("c")
        s = lax.axis_index("s")
        wid = c * 16 + s
        rpt = NACC // 16
        rows0 = s * rpt
        pltpu.sync_copy(zacc_h.at[pl.ds(rows0, rpt)],
                        acc_sh.at[pl.ds(rows0, rpt)])
        plsc.subcore_barrier()
        base0 = wid * EPW

        @pl.loop(0, NCH)
        def _(j):
            base = base0 + j * CH
            pltpu.sync_copy(gidx_h.at[pl.ds(base, CH)], gi_v)
            pltpu.sync_copy(dst_h.at[pl.ds(base, CH)], di_v)
            pltpu.sync_copy(table_h.at[gi_v], rows_v)
            pltpu.sync_copy(rows_v, acc_sh.at[di_v], add=True)

        plsc.subcore_barrier()
        pltpu.sync_copy(acc_sh.at[pl.ds(rows0, rpt)],
                        out_h.at[c, pl.ds(rows0, rpt)])

    return k(table, gidx_p, dsts_p, zacc)


def _padw(a, fill):
    return jnp.pad(a.reshape(NW, E // NW), ((0, 0), (0, EPW - E // NW)),
                   constant_values=fill).reshape(-1)


def _dot(a, b):
    return lax.dot_general(a, b, (((1,), (0,)), ((), ())),
                           precision=lax.Precision.HIGHEST,
                           preferred_element_type=jnp.float32)


def _phase1_body(x_ref, xi_ref, wrel_ref, wself_ref, bgcn_ref, wk_ref, wv_ref,
                 wia_ref, xw_ref, k_ref, v_ref, xself_ref, ctxd_ref):
    xb = x_ref[...]
    ones = jnp.ones((xb.shape[0], TW - D), jnp.float32)
    for r in range(R):
        xw_ref[r] = jnp.concatenate([_dot(xb, wrel_ref[r]), ones], axis=1)
    k_ref[...] = _dot(xb, wk_ref[...])
    v_ref[...] = jnp.concatenate([_dot(xb, wv_ref[...]), ones], axis=1)
    xself_ref[...] = _dot(xb, wself_ref[...]) + bgcn_ref[...]
    ctxd_ref[...] = _dot(xi_ref[...], wia_ref[...])


def _phase1(x, x_init, W_rel, W_self, b_gcn, Wk, Wv, wia):
    grid = (N // BN,)
    full = lambda i: (0, 0)
    out_shapes = (
        jax.ShapeDtypeStruct((R, N, TW), jnp.float32),
        jax.ShapeDtypeStruct((N, D), jnp.float32),
        jax.ShapeDtypeStruct((N, TW), jnp.float32),
        jax.ShapeDtypeStruct((N, D), jnp.float32),
        jax.ShapeDtypeStruct((N, 1), jnp.float32),
    )
    return pl.pallas_call(
        _phase1_body,
        grid=grid,
        in_specs=[
            pl.BlockSpec((BN, D), lambda i: (i, 0)),
            pl.BlockSpec((BN, D), lambda i: (i, 0)),
            pl.BlockSpec((R, D, D), lambda i: (0, 0, 0)),
            pl.BlockSpec((D, D), full),
            pl.BlockSpec((1, D), full),
            pl.BlockSpec((D, D), full),
            pl.BlockSpec((D, D), full),
            pl.BlockSpec((D, 1), full),
        ],
        out_specs=(
            pl.BlockSpec((R, BN, TW), lambda i: (0, i, 0)),
            pl.BlockSpec((BN, D), lambda i: (i, 0)),
            pl.BlockSpec((BN, TW), lambda i: (i, 0)),
            pl.BlockSpec((BN, D), lambda i: (i, 0)),
            pl.BlockSpec((BN, 1), lambda i: (i, 0)),
        ),
        out_shape=out_shapes,
    )(x, x_init, W_rel, W_self, b_gcn.reshape(1, D), Wk, Wv, wia)


def _phase2_body(agg_ref, deg_ref, xself_ref, wq_ref, h_ref, q_ref):
    degc = jnp.maximum(deg_ref[...], 1.0)
    h = agg_ref[...] / degc + xself_ref[...]
    h_ref[...] = h
    q_ref[...] = _dot(h, wq_ref[...])


def _phase2(agg, deg, xself, Wq_s):
    grid = (N // BN,)
    return pl.pallas_call(
        _phase2_body,
        grid=grid,
        in_specs=[
            pl.BlockSpec((BN, D), lambda i: (i, 0)),
            pl.BlockSpec((BN, 1), lambda i: (i, 0)),
            pl.BlockSpec((BN, D), lambda i: (i, 0)),
            pl.BlockSpec((D, D), lambda i: (0, 0)),
        ],
        out_specs=(
            pl.BlockSpec((BN, D), lambda i: (i, 0)),
            pl.BlockSpec((BN, D), lambda i: (i, 0)),
        ),
        out_shape=(
            jax.ShapeDtypeStruct((N, D), jnp.float32),
            jax.ShapeDtypeStruct((N, D), jnp.float32),
        ),
    )(agg, deg.reshape(N, 1), xself, Wq_s)


def _phase3_body(h_ref, cacc_ref, dinv_ref, wlin_ref, blin_ref, z_ref):
    z = h_ref[...] + cacc_ref[...] * dinv_ref[...]
    z_ref[...] = _dot(z, wlin_ref[...]) + blin_ref[...]


def _phase3(h, cacc, dinv, W_lin, b_lin):
    grid = (N // BN,)
    return pl.pallas_call(
        _phase3_body,
        grid=grid,
        in_specs=[
            pl.BlockSpec((BN, D), lambda i: (i, 0)),
            pl.BlockSpec((BN, D), lambda i: (i, 0)),
            pl.BlockSpec((BN, 1), lambda i: (i, 0)),
            pl.BlockSpec((D, D), lambda i: (0, 0)),
            pl.BlockSpec((1, D), lambda i: (0, 0)),
        ],
        out_specs=pl.BlockSpec((BN, D), lambda i: (i, 0)),
        out_shape=jax.ShapeDtypeStruct((N, D), jnp.float32),
    )(h, cacc, dinv.reshape(N, 1), W_lin, b_lin.reshape(1, D))


def kernel(f, x, x_init, edge_index, edge_type, f_batch, x_batch, W_rel,
           W_self, b_gcn, Wq, Wk, Wv, Wf, Wi, a_vec, W_lin, b_lin):
    src = edge_index[0]
    dst = edge_index[1]

    fw = f @ Wf
    f_g = jax.ops.segment_sum(fw, f_batch, num_segments=G)
    f_cnt = jax.ops.segment_sum(jnp.ones((G,), x.dtype), f_batch,
                                num_segments=G)
    f_g = f_g / jnp.clip(f_cnt, 1.0)[:, None]
    fdot = f_g @ a_vec

    wia = (Wi @ a_vec).reshape(D, 1)
    Wq_s = Wq * (1.0 / math.sqrt(D))

    xw, k, v, xself, ctxd = _phase1(x, x_init, W_rel, W_self, b_gcn, Wk, Wv,
                                    wia)
    ctxs = ctxd[:, 0] + fdot[x_batch]

    table = xw.reshape(R * N, TW)
    gidx_p = _padw(edge_type * N + src, 0)
    dsts_p = _padw(dst, N)
    zacc = jnp.zeros((NACC, TW), jnp.float32)
    accs = _sc_pass_a(table, gidx_p, dsts_p, zacc)
    agg = accs[0, :N, :D] + accs[1, :N, :D]
    deg = accs[0, :N, D] + accs[1, :N, D]

    h, q = _phase2(agg, deg, xself, Wq_s)

    score = (jnp.take(q, dst, axis=0) * jnp.take(k, src, axis=0)).sum(-1)
    score = score + jnp.take(ctxs, dst)
    score = jnp.where(score >= 0, score, 0.2 * score)

    smax = jax.ops.segment_max(score, dst, num_segments=N)
    smax = jnp.where(jnp.isfinite(smax), smax, 0.0)

    ex = jnp.exp(score - jnp.take(smax, dst))
    denom = jax.ops.segment_sum(ex, dst, num_segments=N)
    cacc = jax.ops.segment_sum(ex[:, None] * jnp.take(v[:, :D], src, axis=0),
                               dst, num_segments=N)
    dinv = 1.0 / jnp.clip(denom, 1e-16)
    alpha = ex * jnp.take(dinv, dst)

    z = _phase3(h, cacc, dinv, W_lin, b_lin)
    return (z, alpha)


# speedup vs baseline: 2.1548x; 1.4768x over previous
import functools
import math

import jax
import jax.numpy as jnp
from jax import lax
from jax.experimental import pallas as pl
from jax.experimental.pallas import tpu as pltpu
from jax.experimental.pallas import tpu_sc as plsc

N = 10000
E = 320000
G = 16
R = 8
D = 128

BN = 1000

NW = 32
CH = 128
EPW = 10112
NCH = EPW // CH
EPAD = NW * EPW
NACC = 10240
TW = 144

def _mesh():
    return plsc.VectorSubcoreMesh(core_axis_name="c", subcore_axis_name="s")


def _sc_params(layout_passes=True):
    kw = dict(use_tc_tiling_on_sc=False)
    if not layout_passes:
        kw["needs_layout_passes"] = False
    return pltpu.CompilerParams(**kw)


def _sc_pass_a(table, gidx_p, dsts_p, zacc):
    @functools.partial(
        pl.kernel,
        mesh=_mesh(),
        compiler_params=pltpu.CompilerParams(use_tc_tiling_on_sc=False),
        out_type=jax.ShapeDtypeStruct((2, NACC, TW), jnp.float32),
        scratch_types=[
            pltpu.VMEM((CH,), jnp.int32),
            pltpu.VMEM((CH,), jnp.int32),
            pltpu.VMEM((CH, TW), jnp.float32),
            pltpu.VMEM_SHARED((NACC, TW), jnp.float32),
        ],
    )
    def k(table_h, gidx_h, dst_h, zacc_h, out_h, gi_v, di_v, rows_v, acc_sh):
        c = lax.axis_index("c")
        s = lax.axis_index("s")
        wid = c * 16 + s
        rpt = NACC // 16
        rows0 = s * rpt
        pltpu.sync_copy(zacc_h.at[pl.ds(rows0, rpt)],
                        acc_sh.at[pl.ds(rows0, rpt)])
        plsc.subcore_barrier()
        base0 = wid * EPW

        @pl.loop(0, NCH)
        def _(j):
            base = base0 + j * CH
            pltpu.sync_copy(gidx_h.at[pl.ds(base, CH)], gi_v)
            pltpu.sync_copy(dst_h.at[pl.ds(base, CH)], di_v)
            pltpu.sync_copy(table_h.at[gi_v], rows_v)
            pltpu.sync_copy(rows_v, acc_sh.at[di_v], add=True)

        plsc.subcore_barrier()
        pltpu.sync_copy(acc_sh.at[pl.ds(rows0, rpt)],
                        out_h.at[c, pl.ds(rows0, rpt)])

    return k(table, gidx_p, dsts_p, zacc)


def _sc_pass_b(q, kt, ctxs_pad, dstg_p, src_p, mask_p):
    @functools.partial(
        pl.kernel,
        mesh=_mesh(),
        compiler_params=_sc_params(layout_passes=False),
        out_type=(
            jax.ShapeDtypeStruct((EPAD,), jnp.float32),
            jax.ShapeDtypeStruct((NW, N), jnp.float32),
        ),
        scratch_types=[
            pltpu.VMEM((CH,), jnp.int32),
            pltpu.VMEM((CH,), jnp.int32),
            pltpu.VMEM((CH,), jnp.float32),
            pltpu.VMEM((CH, D), jnp.float32),
            pltpu.VMEM((CH, D), jnp.float32),
            pltpu.VMEM((CH,), jnp.float32),
            pltpu.VMEM((NACC,), jnp.float32),
            pltpu.VMEM((N,), jnp.float32),
        ],
    )
    def k(q_h, k_h, ctxs_h, dstg_h, src_h, mask_h, sco_h, mx_h,
          di_v, si_v, mk_v, qbuf, kbuf, sc_v, ctxs_v, mx_v):
        c = lax.axis_index("c")
        s = lax.axis_index("s")
        wid = c * 16 + s
        pltpu.sync_copy(ctxs_h, ctxs_v)
        neg_inf = jnp.full((16,), -jnp.inf, jnp.float32)

        @pl.loop(0, N, step=16)
        def _(i):
            mx_v[pl.ds(i, 16)] = neg_inf

        base0 = wid * EPW

        @pl.loop(0, NCH)
        def _(j):
            base = base0 + j * CH
            pltpu.sync_copy(dstg_h.at[pl.ds(base, CH)], di_v)
            pltpu.sync_copy(src_h.at[pl.ds(base, CH)], si_v)
            pltpu.sync_copy(mask_h.at[pl.ds(base, CH)], mk_v)
            pltpu.sync_copy(q_h.at[di_v], qbuf)
            pltpu.sync_copy(k_h.at[si_v], kbuf)

            @pl.loop(0, CH, step=16)
            def _(e0):
                rows = e0 + lax.iota(jnp.int32, 16)

                def dbody(d, acc):
                    col = jnp.full((16,), d, jnp.int32)
                    qv = plsc.load_gather(qbuf, [rows, col])
                    kv = plsc.load_gather(kbuf, [rows, col])
                    return acc + qv * kv

                acc = lax.fori_loop(0, D, dbody,
                                    jnp.zeros((16,), jnp.float32),
                                    unroll=16)
                di16 = di_v[pl.ds(e0, 16)]
                s16 = acc + plsc.load_gather(ctxs_v, [di16])
                s16 = jnp.where(s16 >= 0, s16, 0.2 * s16)
                s16 = jnp.where(mk_v[pl.ds(e0, 16)] > 0, s16, neg_inf)
                sc_v[pl.ds(e0, 16)] = s16

                def mx_body(_):
                    m = plsc.load_gather(mx_v, [di16])
                    need = s16 > m
                    plsc.store_scatter(mx_v, [di16], s16, mask=need)
                    m2 = plsc.load_gather(mx_v, [di16])
                    return jnp.any(s16 > m2)

                lax.while_loop(lambda cont: cont, mx_body, jnp.bool_(True))

            pltpu.sync_copy(sc_v, sco_h.at[pl.ds(base, CH)])

        pltpu.sync_copy(mx_v, mx_h.at[wid])

    return k(q, kt, ctxs_pad, dstg_p, src_p, mask_p)


def _padw(a, fill):
    return jnp.pad(a.reshape(NW, E // NW), ((0, 0), (0, EPW - E // NW)),
                   constant_values=fill).reshape(-1)


def _dot(a, b):
    return lax.dot_general(a, b, (((1,), (0,)), ((), ())),
                           precision=lax.Precision.HIGHEST,
                           preferred_element_type=jnp.float32)


def _phase1_body(x_ref, xi_ref, wrel_ref, wself_ref, bgcn_ref, wk_ref, wv_ref,
                 wia_ref, xw_ref, k_ref, v_ref, xself_ref, ctxd_ref):
    xb = x_ref[...]
    ones = jnp.ones((xb.shape[0], TW - D), jnp.float32)
    for r in range(R):
        xw_ref[r] = jnp.concatenate([_dot(xb, wrel_ref[r]), ones], axis=1)
    k_ref[...] = _dot(xb, wk_ref[...])
    v_ref[...] = jnp.concatenate([_dot(xb, wv_ref[...]), ones], axis=1)
    xself_ref[...] = _dot(xb, wself_ref[...]) + bgcn_ref[...]
    ctxd_ref[...] = _dot(xi_ref[...], wia_ref[...])


def _phase1(x, x_init, W_rel, W_self, b_gcn, Wk, Wv, wia):
    grid = (N // BN,)
    full = lambda i: (0, 0)
    out_shapes = (
        jax.ShapeDtypeStruct((R, N, TW), jnp.float32),
        jax.ShapeDtypeStruct((N, D), jnp.float32),
        jax.ShapeDtypeStruct((N, TW), jnp.float32),
        jax.ShapeDtypeStruct((N, D), jnp.float32),
        jax.ShapeDtypeStruct((N, 1), jnp.float32),
    )
    return pl.pallas_call(
        _phase1_body,
        grid=grid,
        in_specs=[
            pl.BlockSpec((BN, D), lambda i: (i, 0)),
            pl.BlockSpec((BN, D), lambda i: (i, 0)),
            pl.BlockSpec((R, D, D), lambda i: (0, 0, 0)),
            pl.BlockSpec((D, D), full),
            pl.BlockSpec((1, D), full),
            pl.BlockSpec((D, D), full),
            pl.BlockSpec((D, D), full),
            pl.BlockSpec((D, 1), full),
        ],
        out_specs=(
            pl.BlockSpec((R, BN, TW), lambda i: (0, i, 0)),
            pl.BlockSpec((BN, D), lambda i: (i, 0)),
            pl.BlockSpec((BN, TW), lambda i: (i, 0)),
            pl.BlockSpec((BN, D), lambda i: (i, 0)),
            pl.BlockSpec((BN, 1), lambda i: (i, 0)),
        ),
        out_shape=out_shapes,
    )(x, x_init, W_rel, W_self, b_gcn.reshape(1, D), Wk, Wv, wia)


def _phase2_body(agg_ref, deg_ref, xself_ref, wq_ref, h_ref, q_ref):
    degc = jnp.maximum(deg_ref[...], 1.0)
    h = agg_ref[...] / degc + xself_ref[...]
    h_ref[...] = h
    q_ref[...] = _dot(h, wq_ref[...])


def _phase2(agg, deg, xself, Wq_s):
    grid = (N // BN,)
    return pl.pallas_call(
        _phase2_body,
        grid=grid,
        in_specs=[
            pl.BlockSpec((BN, D), lambda i: (i, 0)),
            pl.BlockSpec((BN, 1), lambda i: (i, 0)),
            pl.BlockSpec((BN, D), lambda i: (i, 0)),
            pl.BlockSpec((D, D), lambda i: (0, 0)),
        ],
        out_specs=(
            pl.BlockSpec((BN, D), lambda i: (i, 0)),
            pl.BlockSpec((BN, D), lambda i: (i, 0)),
        ),
        out_shape=(
            jax.ShapeDtypeStruct((N, D), jnp.float32),
            jax.ShapeDtypeStruct((N, D), jnp.float32),
        ),
    )(agg, deg.reshape(N, 1), xself, Wq_s)


def _phase3_body(h_ref, cacc_ref, dinv_ref, wlin_ref, blin_ref, z_ref):
    z = h_ref[...] + cacc_ref[...] * dinv_ref[...]
    z_ref[...] = _dot(z, wlin_ref[...]) + blin_ref[...]


def _phase3(h, cacc, dinv, W_lin, b_lin):
    grid = (N // BN,)
    return pl.pallas_call(
        _phase3_body,
        grid=grid,
        in_specs=[
            pl.BlockSpec((BN, D), lambda i: (i, 0)),
            pl.BlockSpec((BN, D), lambda i: (i, 0)),
            pl.BlockSpec((BN, 1), lambda i: (i, 0)),
            pl.BlockSpec((D, D), lambda i: (0, 0)),
            pl.BlockSpec((1, D), lambda i: (0, 0)),
        ],
        out_specs=pl.BlockSpec((BN, D), lambda i: (i, 0)),
        out_shape=jax.ShapeDtypeStruct((N, D), jnp.float32),
    )(h, cacc, dinv.reshape(N, 1), W_lin, b_lin.reshape(1, D))


def kernel(f, x, x_init, edge_index, edge_type, f_batch, x_batch, W_rel,
           W_self, b_gcn, Wq, Wk, Wv, Wf, Wi, a_vec, W_lin, b_lin):
    src = edge_index[0]
    dst = edge_index[1]

    fw = f @ Wf
    f_g = jax.ops.segment_sum(fw, f_batch, num_segments=G)
    f_cnt = jax.ops.segment_sum(jnp.ones((G,), x.dtype), f_batch,
                                num_segments=G)
    f_g = f_g / jnp.clip(f_cnt, 1.0)[:, None]
    fdot = f_g @ a_vec

    wia = (Wi @ a_vec).reshape(D, 1)
    Wq_s = Wq * (1.0 / math.sqrt(D))

    xw, k, v, xself, ctxd = _phase1(x, x_init, W_rel, W_self, b_gcn, Wk, Wv,
                                    wia)
    ctxs = ctxd[:, 0] + jax.nn.one_hot(x_batch, G, dtype=jnp.float32) @ fdot

    table = xw.reshape(R * N, TW)
    gidx_p = _padw(edge_type * N + src, 0)
    dsts_p = _padw(dst, N)
    zacc = jnp.zeros((NACC, TW), jnp.float32)
    accs = _sc_pass_a(table, gidx_p, dsts_p, zacc)
    agg = accs[0, :N, :D] + accs[1, :N, :D]
    deg = accs[0, :N, D] + accs[1, :N, D]

    h, q = _phase2(agg, deg, xself, Wq_s)

    dstg_p = _padw(dst, 0)
    src_p = _padw(src, 0)
    mask_p = _padw(jnp.ones((E,), jnp.float32), 0.0)
    ctxs_pad = jnp.pad(ctxs, (0, NACC - N))
    scores_p, mx = _sc_pass_b(q, k, ctxs_pad, dstg_p, src_p, mask_p)
    score = scores_p.reshape(NW, EPW)[:, :E // NW].reshape(E)

    smax = jnp.max(mx, axis=0)
    smax = jnp.where(jnp.isfinite(smax), smax, 0.0)

    ex = jnp.exp(score - jnp.take(smax, dst))
    denom = jax.ops.segment_sum(ex, dst, num_segments=N)
    cacc = jax.ops.segment_sum(ex[:, None] * jnp.take(v[:, :D], src, axis=0),
                               dst, num_segments=N)
    dinv = 1.0 / jnp.clip(denom, 1e-16)
    alpha = ex * jnp.take(dinv, dst)

    z = _phase3(h, cacc, dinv, W_lin, b_lin)
    return (z, alpha)


# speedup vs baseline: 3.5551x; 1.6499x over previous
import functools
import math

import jax
import jax.numpy as jnp
from jax import lax
from jax.experimental import pallas as pl
from jax.experimental.pallas import tpu as pltpu
from jax.experimental.pallas import tpu_sc as plsc

N = 10000
E = 320000
G = 16
R = 8
D = 128

BN = 1000

NW = 32
CH = 128
EPW = 10112
NCH = EPW // CH
EPAD = NW * EPW
NACC = 10240
TW = 144

def _mesh():
    return plsc.VectorSubcoreMesh(core_axis_name="c", subcore_axis_name="s")


def _sc_params(layout_passes=True):
    kw = dict(use_tc_tiling_on_sc=False)
    if not layout_passes:
        kw["needs_layout_passes"] = False
    return pltpu.CompilerParams(**kw)


def _sc_pass_a(table, gidx_p, dsts_p, zacc):
    @functools.partial(
        pl.kernel,
        mesh=_mesh(),
        compiler_params=pltpu.CompilerParams(use_tc_tiling_on_sc=False),
        out_type=jax.ShapeDtypeStruct((2, NACC, TW), jnp.float32),
        scratch_types=[
            pltpu.VMEM((CH,), jnp.int32),
            pltpu.VMEM((CH,), jnp.int32),
            pltpu.VMEM((CH, TW), jnp.float32),
            pltpu.VMEM_SHARED((NACC, TW), jnp.float32),
        ],
    )
    def k(table_h, gidx_h, dst_h, zacc_h, out_h, gi_v, di_v, rows_v, acc_sh):
        c = lax.axis_index("c")
        s = lax.axis_index("s")
        wid = c * 16 + s
        rpt = NACC // 16
        rows0 = s * rpt
        pltpu.sync_copy(zacc_h.at[pl.ds(rows0, rpt)],
                        acc_sh.at[pl.ds(rows0, rpt)])
        plsc.subcore_barrier()
        base0 = wid * EPW

        @pl.loop(0, NCH)
        def _(j):
            base = base0 + j * CH
            pltpu.sync_copy(gidx_h.at[pl.ds(base, CH)], gi_v)
            pltpu.sync_copy(dst_h.at[pl.ds(base, CH)], di_v)
            pltpu.sync_copy(table_h.at[gi_v], rows_v)
            pltpu.sync_copy(rows_v, acc_sh.at[di_v], add=True)

        plsc.subcore_barrier()
        pltpu.sync_copy(acc_sh.at[pl.ds(rows0, rpt)],
                        out_h.at[c, pl.ds(rows0, rpt)])

    return k(table, gidx_p, dsts_p, zacc)


def _sc_pass_b(q, kt, ctxs_pad, dstg_p, src_p, mask_p):
    @functools.partial(
        pl.kernel,
        mesh=_mesh(),
        compiler_params=_sc_params(layout_passes=False),
        out_type=(
            jax.ShapeDtypeStruct((EPAD,), jnp.float32),
            jax.ShapeDtypeStruct((NW, N), jnp.float32),
        ),
        scratch_types=[
            pltpu.VMEM((CH,), jnp.int32),
            pltpu.VMEM((CH,), jnp.int32),
            pltpu.VMEM((CH,), jnp.float32),
            pltpu.VMEM((CH, D), jnp.float32),
            pltpu.VMEM((CH, D), jnp.float32),
            pltpu.VMEM((CH,), jnp.float32),
            pltpu.VMEM((NACC,), jnp.float32),
            pltpu.VMEM((N,), jnp.float32),
        ],
    )
    def k(q_h, k_h, ctxs_h, dstg_h, src_h, mask_h, sco_h, mx_h,
          di_v, si_v, mk_v, qbuf, kbuf, sc_v, ctxs_v, mx_v):
        c = lax.axis_index("c")
        s = lax.axis_index("s")
        wid = c * 16 + s
        pltpu.sync_copy(ctxs_h, ctxs_v)
        neg_inf = jnp.full((16,), -jnp.inf, jnp.float32)

        @pl.loop(0, N, step=16)
        def _(i):
            mx_v[pl.ds(i, 16)] = neg_inf

        base0 = wid * EPW

        @pl.loop(0, NCH)
        def _(j):
            base = base0 + j * CH
            pltpu.sync_copy(dstg_h.at[pl.ds(base, CH)], di_v)
            pltpu.sync_copy(src_h.at[pl.ds(base, CH)], si_v)
            pltpu.sync_copy(mask_h.at[pl.ds(base, CH)], mk_v)
            pltpu.sync_copy(q_h.at[di_v], qbuf)
            pltpu.sync_copy(k_h.at[si_v], kbuf)

            @pl.loop(0, CH, step=16)
            def _(e0):
                rows = e0 + lax.iota(jnp.int32, 16)

                def dbody(d, acc):
                    col = jnp.full((16,), d, jnp.int32)
                    qv = plsc.load_gather(qbuf, [rows, col])
                    kv = plsc.load_gather(kbuf, [rows, col])
                    return acc + qv * kv

                acc = lax.fori_loop(0, D, dbody,
                                    jnp.zeros((16,), jnp.float32),
                                    unroll=16)
                di16 = di_v[pl.ds(e0, 16)]
                s16 = acc + plsc.load_gather(ctxs_v, [di16])
                s16 = jnp.where(s16 >= 0, s16, 0.2 * s16)
                s16 = jnp.where(mk_v[pl.ds(e0, 16)] > 0, s16, neg_inf)
                sc_v[pl.ds(e0, 16)] = s16

                def mx_body(_):
                    m = plsc.load_gather(mx_v, [di16])
                    need = s16 > m
                    plsc.store_scatter(mx_v, [di16], s16, mask=need)
                    m2 = plsc.load_gather(mx_v, [di16])
                    return jnp.any(s16 > m2)

                lax.while_loop(lambda cont: cont, mx_body, jnp.bool_(True))

            pltpu.sync_copy(sc_v, sco_h.at[pl.ds(base, CH)])

        pltpu.sync_copy(mx_v, mx_h.at[wid])

    return k(q, kt, ctxs_pad, dstg_p, src_p, mask_p)


def _sc_pass_c(vt, smax_pad, scores_p, dstg_p, dsts_p, src_p, zacc):
    @functools.partial(
        pl.kernel,
        mesh=_mesh(),
        compiler_params=_sc_params(layout_passes=False),
        out_type=(
            jax.ShapeDtypeStruct((2, NACC, TW), jnp.float32),
            jax.ShapeDtypeStruct((EPAD,), jnp.float32),
        ),
        scratch_types=[
            pltpu.VMEM((CH,), jnp.int32),
            pltpu.VMEM((CH,), jnp.int32),
            pltpu.VMEM((CH,), jnp.int32),
            pltpu.VMEM((CH,), jnp.float32),
            pltpu.VMEM((CH,), jnp.float32),
            pltpu.VMEM((CH, TW), jnp.float32),
            pltpu.VMEM((NACC,), jnp.float32),
            pltpu.VMEM_SHARED((NACC, TW), jnp.float32),
        ],
    )
    def k(vt_h, smax_h, sco_h, dstg_h, dsts_h, src_h, zacc_h, out_h, exo_h,
          di_v, ds_v, si_v, sc_v, ex_v, vbuf, smax_v, acc_sh):
        c = lax.axis_index("c")
        s = lax.axis_index("s")
        wid = c * 16 + s
        rpt = NACC // 16
        rows0 = s * rpt
        pltpu.sync_copy(zacc_h.at[pl.ds(rows0, rpt)],
                        acc_sh.at[pl.ds(rows0, rpt)])
        pltpu.sync_copy(smax_h, smax_v)
        plsc.subcore_barrier()
        base0 = wid * EPW

        @pl.loop(0, NCH)
        def _(j):
            base = base0 + j * CH
            pltpu.sync_copy(dstg_h.at[pl.ds(base, CH)], di_v)
            pltpu.sync_copy(dsts_h.at[pl.ds(base, CH)], ds_v)
            pltpu.sync_copy(src_h.at[pl.ds(base, CH)], si_v)
            pltpu.sync_copy(sco_h.at[pl.ds(base, CH)], sc_v)
            pltpu.sync_copy(vt_h.at[si_v], vbuf)

            @pl.loop(0, CH, step=16)
            def _(e0):
                rows = e0 + lax.iota(jnp.int32, 16)
                di16 = di_v[pl.ds(e0, 16)]
                sm16 = plsc.load_gather(smax_v, [di16])
                ex16 = jnp.exp(sc_v[pl.ds(e0, 16)] - sm16)
                ex_v[pl.ds(e0, 16)] = ex16

                def dbody(d, carry):
                    col = jnp.full((16,), d, jnp.int32)
                    val = plsc.load_gather(vbuf, [rows, col]) * ex16
                    plsc.store_scatter(vbuf, [rows, col], val)
                    return carry

                lax.fori_loop(0, TW, dbody, 0, unroll=16)

            pltpu.sync_copy(vbuf, acc_sh.at[ds_v], add=True)
            pltpu.sync_copy(ex_v, exo_h.at[pl.ds(base, CH)])

        plsc.subcore_barrier()
        pltpu.sync_copy(acc_sh.at[pl.ds(rows0, rpt)],
                        out_h.at[c, pl.ds(rows0, rpt)])

    return k(vt, smax_pad, scores_p, dstg_p, dsts_p, src_p, zacc)


def _sc_pass_d(ex_p, dstg_p, dinv_pad):
    @functools.partial(
        pl.kernel,
        mesh=_mesh(),
        compiler_params=_sc_params(layout_passes=False),
        out_type=jax.ShapeDtypeStruct((EPAD,), jnp.float32),
        scratch_types=[
            pltpu.VMEM((CH,), jnp.int32),
            pltpu.VMEM((CH,), jnp.float32),
            pltpu.VMEM((CH,), jnp.float32),
            pltpu.VMEM((NACC,), jnp.float32),
        ],
    )
    def k(ex_h, dstg_h, dinv_h, al_h, di_v, ex_v, al_v, dinv_v):
        c = lax.axis_index("c")
        s = lax.axis_index("s")
        wid = c * 16 + s
        pltpu.sync_copy(dinv_h, dinv_v)
        base0 = wid * EPW

        @pl.loop(0, NCH)
        def _(j):
            base = base0 + j * CH
            pltpu.sync_copy(dstg_h.at[pl.ds(base, CH)], di_v)
            pltpu.sync_copy(ex_h.at[pl.ds(base, CH)], ex_v)

            @pl.loop(0, CH, step=16)
            def _(e0):
                di16 = di_v[pl.ds(e0, 16)]
                al_v[pl.ds(e0, 16)] = (ex_v[pl.ds(e0, 16)]
                                       * plsc.load_gather(dinv_v, [di16]))

            pltpu.sync_copy(al_v, al_h.at[pl.ds(base, CH)])

    return k(ex_p, dstg_p, dinv_pad)


def _padw(a, fill):
    return jnp.pad(a.reshape(NW, E // NW), ((0, 0), (0, EPW - E // NW)),
                   constant_values=fill).reshape(-1)


def _dot(a, b):
    return lax.dot_general(a, b, (((1,), (0,)), ((), ())),
                           precision=lax.Precision.HIGHEST,
                           preferred_element_type=jnp.float32)


def _phase1_body(x_ref, xi_ref, wrel_ref, wself_ref, bgcn_ref, wk_ref, wv_ref,
                 wia_ref, xw_ref, k_ref, v_ref, xself_ref, ctxd_ref):
    xb = x_ref[...]
    ones = jnp.ones((xb.shape[0], TW - D), jnp.float32)
    for r in range(R):
        xw_ref[r] = jnp.concatenate([_dot(xb, wrel_ref[r]), ones], axis=1)
    k_ref[...] = _dot(xb, wk_ref[...])
    v_ref[...] = jnp.concatenate([_dot(xb, wv_ref[...]), ones], axis=1)
    xself_ref[...] = _dot(xb, wself_ref[...]) + bgcn_ref[...]
    ctxd_ref[...] = _dot(xi_ref[...], wia_ref[...])


def _phase1(x, x_init, W_rel, W_self, b_gcn, Wk, Wv, wia):
    grid = (N // BN,)
    full = lambda i: (0, 0)
    out_shapes = (
        jax.ShapeDtypeStruct((R, N, TW), jnp.float32),
        jax.ShapeDtypeStruct((N, D), jnp.float32),
        jax.ShapeDtypeStruct((N, TW), jnp.float32),
        jax.ShapeDtypeStruct((N, D), jnp.float32),
        jax.ShapeDtypeStruct((N, 1), jnp.float32),
    )
    return pl.pallas_call(
        _phase1_body,
        grid=grid,
        in_specs=[
            pl.BlockSpec((BN, D), lambda i: (i, 0)),
            pl.BlockSpec((BN, D), lambda i: (i, 0)),
            pl.BlockSpec((R, D, D), lambda i: (0, 0, 0)),
            pl.BlockSpec((D, D), full),
            pl.BlockSpec((1, D), full),
            pl.BlockSpec((D, D), full),
            pl.BlockSpec((D, D), full),
            pl.BlockSpec((D, 1), full),
        ],
        out_specs=(
            pl.BlockSpec((R, BN, TW), lambda i: (0, i, 0)),
            pl.BlockSpec((BN, D), lambda i: (i, 0)),
            pl.BlockSpec((BN, TW), lambda i: (i, 0)),
            pl.BlockSpec((BN, D), lambda i: (i, 0)),
            pl.BlockSpec((BN, 1), lambda i: (i, 0)),
        ),
        out_shape=out_shapes,
    )(x, x_init, W_rel, W_self, b_gcn.reshape(1, D), Wk, Wv, wia)


def _phase2_body(agg_ref, deg_ref, xself_ref, wq_ref, h_ref, q_ref):
    degc = jnp.maximum(deg_ref[...], 1.0)
    h = agg_ref[...] / degc + xself_ref[...]
    h_ref[...] = h
    q_ref[...] = _dot(h, wq_ref[...])


def _phase2(agg, deg, xself, Wq_s):
    grid = (N // BN,)
    return pl.pallas_call(
        _phase2_body,
        grid=grid,
        in_specs=[
            pl.BlockSpec((BN, D), lambda i: (i, 0)),
            pl.BlockSpec((BN, 1), lambda i: (i, 0)),
            pl.BlockSpec((BN, D), lambda i: (i, 0)),
            pl.BlockSpec((D, D), lambda i: (0, 0)),
        ],
        out_specs=(
            pl.BlockSpec((BN, D), lambda i: (i, 0)),
            pl.BlockSpec((BN, D), lambda i: (i, 0)),
        ),
        out_shape=(
            jax.ShapeDtypeStruct((N, D), jnp.float32),
            jax.ShapeDtypeStruct((N, D), jnp.float32),
        ),
    )(agg, deg.reshape(N, 1), xself, Wq_s)


def _phase3_body(h_ref, cacc_ref, dinv_ref, wlin_ref, blin_ref, z_ref):
    z = h_ref[...] + cacc_ref[...] * dinv_ref[...]
    z_ref[...] = _dot(z, wlin_ref[...]) + blin_ref[...]


def _phase3(h, cacc, dinv, W_lin, b_lin):
    grid = (N // BN,)
    return pl.pallas_call(
        _phase3_body,
        grid=grid,
        in_specs=[
            pl.BlockSpec((BN, D), lambda i: (i, 0)),
            pl.BlockSpec((BN, D), lambda i: (i, 0)),
            pl.BlockSpec((BN, 1), lambda i: (i, 0)),
            pl.BlockSpec((D, D), lambda i: (0, 0)),
            pl.BlockSpec((1, D), lambda i: (0, 0)),
        ],
        out_specs=pl.BlockSpec((BN, D), lambda i: (i, 0)),
        out_shape=jax.ShapeDtypeStruct((N, D), jnp.float32),
    )(h, cacc, dinv.reshape(N, 1), W_lin, b_lin.reshape(1, D))


def kernel(f, x, x_init, edge_index, edge_type, f_batch, x_batch, W_rel,
           W_self, b_gcn, Wq, Wk, Wv, Wf, Wi, a_vec, W_lin, b_lin):
    src = edge_index[0]
    dst = edge_index[1]

    fw = f @ Wf
    f_g = jax.ops.segment_sum(fw, f_batch, num_segments=G)
    f_cnt = jax.ops.segment_sum(jnp.ones((G,), x.dtype), f_batch,
                                num_segments=G)
    f_g = f_g / jnp.clip(f_cnt, 1.0)[:, None]
    fdot = f_g @ a_vec

    wia = (Wi @ a_vec).reshape(D, 1)
    Wq_s = Wq * (1.0 / math.sqrt(D))

    xw, k, v, xself, ctxd = _phase1(x, x_init, W_rel, W_self, b_gcn, Wk, Wv,
                                    wia)
    ctxs = ctxd[:, 0] + jax.nn.one_hot(x_batch, G, dtype=jnp.float32) @ fdot

    table = xw.reshape(R * N, TW)
    gidx_p = _padw(edge_type * N + src, 0)
    dsts_p = _padw(dst, N)
    zacc = jnp.zeros((NACC, TW), jnp.float32)
    accs = _sc_pass_a(table, gidx_p, dsts_p, zacc)
    agg = accs[0, :N, :D] + accs[1, :N, :D]
    deg = accs[0, :N, D] + accs[1, :N, D]

    h, q = _phase2(agg, deg, xself, Wq_s)

    dstg_p = _padw(dst, 0)
    src_p = _padw(src, 0)
    mask_p = _padw(jnp.ones((E,), jnp.float32), 0.0)
    ctxs_pad = jnp.pad(ctxs, (0, NACC - N))
    scores_p, mx = _sc_pass_b(q, k, ctxs_pad, dstg_p, src_p, mask_p)
    score = scores_p.reshape(NW, EPW)[:, :E // NW].reshape(E)

    smax = jnp.max(mx, axis=0)
    smax = jnp.where(jnp.isfinite(smax), smax, 0.0)
    smax_pad = jnp.pad(smax, (0, NACC - N))

    caccs, ex_p = _sc_pass_c(v, smax_pad, scores_p, dstg_p, dsts_p, src_p,
                             zacc)
    cacc = caccs[0, :N, :D] + caccs[1, :N, :D]
    denom = caccs[0, :N, D] + caccs[1, :N, D]
    dinv = 1.0 / jnp.clip(denom, 1e-16)

    dinv_pad = jnp.pad(dinv, (0, NACC - N))
    alpha_p = _sc_pass_d(ex_p, dstg_p, dinv_pad)
    alpha = alpha_p.reshape(NW, EPW)[:, :E // NW].reshape(E)

    z = _phase3(h, cacc, dinv, W_lin, b_lin)
    return (z, alpha)


# speedup vs baseline: 6.4527x; 1.8151x over previous
import functools
import math

import jax
import jax.numpy as jnp
from jax import lax
from jax.experimental import pallas as pl
from jax.experimental.pallas import tpu as pltpu
from jax.experimental.pallas import tpu_sc as plsc

N = 10000
E = 320000
G = 16
R = 8
D = 128

BN = 1000

NW = 32
CH = 128
EPW = 10112
NCH = EPW // CH
EPAD = NW * EPW
NACC = 10240
TW = 144

def _mesh():
    return plsc.VectorSubcoreMesh(core_axis_name="c", subcore_axis_name="s")


def _sc_params(layout_passes=True):
    kw = dict(use_tc_tiling_on_sc=False)
    if not layout_passes:
        kw["needs_layout_passes"] = False
    return pltpu.CompilerParams(**kw)


def _sc_pass_a(table, gidx_p, dsts_p, zacc):
    @functools.partial(
        pl.kernel,
        mesh=_mesh(),
        compiler_params=pltpu.CompilerParams(use_tc_tiling_on_sc=False),
        out_type=jax.ShapeDtypeStruct((2, NACC, TW), jnp.float32),
        scratch_types=[
            pltpu.VMEM((CH,), jnp.int32),
            pltpu.VMEM((CH,), jnp.int32),
            pltpu.VMEM((CH, TW), jnp.float32),
            pltpu.VMEM_SHARED((NACC, TW), jnp.float32),
        ],
    )
    def k(table_h, gidx_h, dst_h, zacc_h, out_h, gi_v, di_v, rows_v, acc_sh):
        c = lax.axis_index("c")
        s = lax.axis_index("s")
        wid = c * 16 + s
        rpt = NACC // 16
        rows0 = s * rpt
        pltpu.sync_copy(zacc_h.at[pl.ds(rows0, rpt)],
                        acc_sh.at[pl.ds(rows0, rpt)])
        plsc.subcore_barrier()
        base0 = wid * EPW

        @pl.loop(0, NCH)
        def _(j):
            base = base0 + j * CH
            pltpu.sync_copy(gidx_h.at[pl.ds(base, CH)], gi_v)
            pltpu.sync_copy(dst_h.at[pl.ds(base, CH)], di_v)
            pltpu.sync_copy(table_h.at[gi_v], rows_v)
            pltpu.sync_copy(rows_v, acc_sh.at[di_v], add=True)

        plsc.subcore_barrier()
        pltpu.sync_copy(acc_sh.at[pl.ds(rows0, rpt)],
                        out_h.at[c, pl.ds(rows0, rpt)])

    return k(table, gidx_p, dsts_p, zacc)


def _sc_pass_b(q, kt, ctxs_pad, dstg_p, src_p, mask_p):
    @functools.partial(
        pl.kernel,
        mesh=_mesh(),
        compiler_params=_sc_params(layout_passes=False),
        out_type=(
            jax.ShapeDtypeStruct((EPAD,), jnp.float32),
            jax.ShapeDtypeStruct((NW, N), jnp.float32),
        ),
        scratch_types=[
            pltpu.VMEM((CH,), jnp.int32),
            pltpu.VMEM((CH,), jnp.int32),
            pltpu.VMEM((CH,), jnp.float32),
            pltpu.VMEM((CH, D), jnp.float32),
            pltpu.VMEM((CH, D), jnp.float32),
            pltpu.VMEM((CH,), jnp.float32),
            pltpu.VMEM((NACC,), jnp.float32),
            pltpu.VMEM((N,), jnp.float32),
        ],
    )
    def k(q_h, k_h, ctxs_h, dstg_h, src_h, mask_h, sco_h, mx_h,
          di_v, si_v, mk_v, qbuf, kbuf, sc_v, ctxs_v, mx_v):
        c = lax.axis_index("c")
        s = lax.axis_index("s")
        wid = c * 16 + s
        pltpu.sync_copy(ctxs_h, ctxs_v)
        neg_inf = jnp.full((16,), -jnp.inf, jnp.float32)

        @pl.loop(0, N, step=16)
        def _(i):
            mx_v[pl.ds(i, 16)] = neg_inf

        base0 = wid * EPW

        @pl.loop(0, NCH)
        def _(j):
            base = base0 + j * CH
            pltpu.sync_copy(dstg_h.at[pl.ds(base, CH)], di_v)
            pltpu.sync_copy(src_h.at[pl.ds(base, CH)], si_v)
            pltpu.sync_copy(mask_h.at[pl.ds(base, CH)], mk_v)
            pltpu.sync_copy(q_h.at[di_v], qbuf)
            pltpu.sync_copy(k_h.at[si_v], kbuf)

            @pl.loop(0, CH, step=16)
            def _(e0):
                lanes = lax.iota(jnp.int32, 16)
                acc = jnp.zeros((16,), jnp.float32)
                for e in range(16):
                    p = (qbuf[e0 + e, pl.ds(0, 16)]
                         * kbuf[e0 + e, pl.ds(0, 16)])
                    for b in range(1, D // 16):
                        p = p + (qbuf[e0 + e, pl.ds(b * 16, 16)]
                                 * kbuf[e0 + e, pl.ds(b * 16, 16)])
                    acc = jnp.where(lanes == e, jnp.sum(p), acc)
                di16 = di_v[pl.ds(e0, 16)]
                s16 = acc + plsc.load_gather(ctxs_v, [di16])
                s16 = jnp.where(s16 >= 0, s16, 0.2 * s16)
                s16 = jnp.where(mk_v[pl.ds(e0, 16)] > 0, s16, neg_inf)
                sc_v[pl.ds(e0, 16)] = s16

                def mx_body(_):
                    m = plsc.load_gather(mx_v, [di16])
                    need = s16 > m
                    plsc.store_scatter(mx_v, [di16], s16, mask=need)
                    m2 = plsc.load_gather(mx_v, [di16])
                    return jnp.any(s16 > m2)

                lax.while_loop(lambda cont: cont, mx_body, jnp.bool_(True))

            pltpu.sync_copy(sc_v, sco_h.at[pl.ds(base, CH)])

        pltpu.sync_copy(mx_v, mx_h.at[wid])

    return k(q, kt, ctxs_pad, dstg_p, src_p, mask_p)


def _sc_pass_c(vt, smax_pad, scores_p, dstg_p, dsts_p, src_p, zacc):
    @functools.partial(
        pl.kernel,
        mesh=_mesh(),
        compiler_params=_sc_params(layout_passes=False),
        out_type=(
            jax.ShapeDtypeStruct((2, NACC, TW), jnp.float32),
            jax.ShapeDtypeStruct((EPAD,), jnp.float32),
        ),
        scratch_types=[
            pltpu.VMEM((CH,), jnp.int32),
            pltpu.VMEM((CH,), jnp.int32),
            pltpu.VMEM((CH,), jnp.int32),
            pltpu.VMEM((CH,), jnp.float32),
            pltpu.VMEM((CH,), jnp.float32),
            pltpu.VMEM((CH, TW), jnp.float32),
            pltpu.VMEM((NACC,), jnp.float32),
            pltpu.VMEM_SHARED((NACC, TW), jnp.float32),
        ],
    )
    def k(vt_h, smax_h, sco_h, dstg_h, dsts_h, src_h, zacc_h, out_h, exo_h,
          di_v, ds_v, si_v, sc_v, ex_v, vbuf, smax_v, acc_sh):
        c = lax.axis_index("c")
        s = lax.axis_index("s")
        wid = c * 16 + s
        rpt = NACC // 16
        rows0 = s * rpt
        pltpu.sync_copy(zacc_h.at[pl.ds(rows0, rpt)],
                        acc_sh.at[pl.ds(rows0, rpt)])
        pltpu.sync_copy(smax_h, smax_v)
        plsc.subcore_barrier()
        base0 = wid * EPW

        @pl.loop(0, NCH)
        def _(j):
            base = base0 + j * CH
            pltpu.sync_copy(dstg_h.at[pl.ds(base, CH)], di_v)
            pltpu.sync_copy(dsts_h.at[pl.ds(base, CH)], ds_v)
            pltpu.sync_copy(src_h.at[pl.ds(base, CH)], si_v)
            pltpu.sync_copy(sco_h.at[pl.ds(base, CH)], sc_v)
            pltpu.sync_copy(vt_h.at[si_v], vbuf)

            @pl.loop(0, CH, step=16)
            def _(e0):
                lanes = lax.iota(jnp.int32, 16)
                di16 = di_v[pl.ds(e0, 16)]
                sm16 = plsc.load_gather(smax_v, [di16])
                ex16 = jnp.exp(sc_v[pl.ds(e0, 16)] - sm16)
                ex_v[pl.ds(e0, 16)] = ex16
                for e in range(16):
                    exs = jnp.sum(jnp.where(lanes == e, ex16, 0.0))
                    for b in range(TW // 16):
                        sl = pl.ds(b * 16, 16)
                        vbuf[e0 + e, sl] = vbuf[e0 + e, sl] * exs

            pltpu.sync_copy(vbuf, acc_sh.at[ds_v], add=True)
            pltpu.sync_copy(ex_v, exo_h.at[pl.ds(base, CH)])

        plsc.subcore_barrier()
        pltpu.sync_copy(acc_sh.at[pl.ds(rows0, rpt)],
                        out_h.at[c, pl.ds(rows0, rpt)])

    return k(vt, smax_pad, scores_p, dstg_p, dsts_p, src_p, zacc)


def _sc_pass_d(ex_p, dstg_p, dinv_pad):
    @functools.partial(
        pl.kernel,
        mesh=_mesh(),
        compiler_params=_sc_params(layout_passes=False),
        out_type=jax.ShapeDtypeStruct((EPAD,), jnp.float32),
        scratch_types=[
            pltpu.VMEM((CH,), jnp.int32),
            pltpu.VMEM((CH,), jnp.float32),
            pltpu.VMEM((CH,), jnp.float32),
            pltpu.VMEM((NACC,), jnp.float32),
        ],
    )
    def k(ex_h, dstg_h, dinv_h, al_h, di_v, ex_v, al_v, dinv_v):
        c = lax.axis_index("c")
        s = lax.axis_index("s")
        wid = c * 16 + s
        pltpu.sync_copy(dinv_h, dinv_v)
        base0 = wid * EPW

        @pl.loop(0, NCH)
        def _(j):
            base = base0 + j * CH
            pltpu.sync_copy(dstg_h.at[pl.ds(base, CH)], di_v)
            pltpu.sync_copy(ex_h.at[pl.ds(base, CH)], ex_v)

            @pl.loop(0, CH, step=16)
            def _(e0):
                di16 = di_v[pl.ds(e0, 16)]
                al_v[pl.ds(e0, 16)] = (ex_v[pl.ds(e0, 16)]
                                       * plsc.load_gather(dinv_v, [di16]))

            pltpu.sync_copy(al_v, al_h.at[pl.ds(base, CH)])

    return k(ex_p, dstg_p, dinv_pad)


def _padw(a, fill):
    return jnp.pad(a.reshape(NW, E // NW), ((0, 0), (0, EPW - E // NW)),
                   constant_values=fill).reshape(-1)


def _dot(a, b):
    return lax.dot_general(a, b, (((1,), (0,)), ((), ())),
                           precision=lax.Precision.HIGHEST,
                           preferred_element_type=jnp.float32)


def _phase1_body(x_ref, xi_ref, wrel_ref, wself_ref, bgcn_ref, wk_ref, wv_ref,
                 wia_ref, xw_ref, k_ref, v_ref, xself_ref, ctxd_ref):
    xb = x_ref[...]
    ones = jnp.ones((xb.shape[0], TW - D), jnp.float32)
    for r in range(R):
        xw_ref[r] = jnp.concatenate([_dot(xb, wrel_ref[r]), ones], axis=1)
    k_ref[...] = _dot(xb, wk_ref[...])
    v_ref[...] = jnp.concatenate([_dot(xb, wv_ref[...]), ones], axis=1)
    xself_ref[...] = _dot(xb, wself_ref[...]) + bgcn_ref[...]
    ctxd_ref[...] = _dot(xi_ref[...], wia_ref[...])


def _phase1(x, x_init, W_rel, W_self, b_gcn, Wk, Wv, wia):
    grid = (N // BN,)
    full = lambda i: (0, 0)
    out_shapes = (
        jax.ShapeDtypeStruct((R, N, TW), jnp.float32),
        jax.ShapeDtypeStruct((N, D), jnp.float32),
        jax.ShapeDtypeStruct((N, TW), jnp.float32),
        jax.ShapeDtypeStruct((N, D), jnp.float32),
        jax.ShapeDtypeStruct((N, 1), jnp.float32),
    )
    return pl.pallas_call(
        _phase1_body,
        grid=grid,
        in_specs=[
            pl.BlockSpec((BN, D), lambda i: (i, 0)),
            pl.BlockSpec((BN, D), lambda i: (i, 0)),
            pl.BlockSpec((R, D, D), lambda i: (0, 0, 0)),
            pl.BlockSpec((D, D), full),
            pl.BlockSpec((1, D), full),
            pl.BlockSpec((D, D), full),
            pl.BlockSpec((D, D), full),
            pl.BlockSpec((D, 1), full),
        ],
        out_specs=(
            pl.BlockSpec((R, BN, TW), lambda i: (0, i, 0)),
            pl.BlockSpec((BN, D), lambda i: (i, 0)),
            pl.BlockSpec((BN, TW), lambda i: (i, 0)),
            pl.BlockSpec((BN, D), lambda i: (i, 0)),
            pl.BlockSpec((BN, 1), lambda i: (i, 0)),
        ),
        out_shape=out_shapes,
    )(x, x_init, W_rel, W_self, b_gcn.reshape(1, D), Wk, Wv, wia)


def _phase2_body(agg_ref, deg_ref, xself_ref, wq_ref, h_ref, q_ref):
    degc = jnp.maximum(deg_ref[...], 1.0)
    h = agg_ref[...] / degc + xself_ref[...]
    h_ref[...] = h
    q_ref[...] = _dot(h, wq_ref[...])


def _phase2(agg, deg, xself, Wq_s):
    grid = (N // BN,)
    return pl.pallas_call(
        _phase2_body,
        grid=grid,
        in_specs=[
            pl.BlockSpec((BN, D), lambda i: (i, 0)),
            pl.BlockSpec((BN, 1), lambda i: (i, 0)),
            pl.BlockSpec((BN, D), lambda i: (i, 0)),
            pl.BlockSpec((D, D), lambda i: (0, 0)),
        ],
        out_specs=(
            pl.BlockSpec((BN, D), lambda i: (i, 0)),
            pl.BlockSpec((BN, D), lambda i: (i, 0)),
        ),
        out_shape=(
            jax.ShapeDtypeStruct((N, D), jnp.float32),
            jax.ShapeDtypeStruct((N, D), jnp.float32),
        ),
    )(agg, deg.reshape(N, 1), xself, Wq_s)


def _phase3_body(h_ref, cacc_ref, dinv_ref, wlin_ref, blin_ref, z_ref):
    z = h_ref[...] + cacc_ref[...] * dinv_ref[...]
    z_ref[...] = _dot(z, wlin_ref[...]) + blin_ref[...]


def _phase3(h, cacc, dinv, W_lin, b_lin):
    grid = (N // BN,)
    return pl.pallas_call(
        _phase3_body,
        grid=grid,
        in_specs=[
            pl.BlockSpec((BN, D), lambda i: (i, 0)),
            pl.BlockSpec((BN, D), lambda i: (i, 0)),
            pl.BlockSpec((BN, 1), lambda i: (i, 0)),
            pl.BlockSpec((D, D), lambda i: (0, 0)),
            pl.BlockSpec((1, D), lambda i: (0, 0)),
        ],
        out_specs=pl.BlockSpec((BN, D), lambda i: (i, 0)),
        out_shape=jax.ShapeDtypeStruct((N, D), jnp.float32),
    )(h, cacc, dinv.reshape(N, 1), W_lin, b_lin.reshape(1, D))


def kernel(f, x, x_init, edge_index, edge_type, f_batch, x_batch, W_rel,
           W_self, b_gcn, Wq, Wk, Wv, Wf, Wi, a_vec, W_lin, b_lin):
    src = edge_index[0]
    dst = edge_index[1]

    fw = f @ Wf
    f_g = jax.ops.segment_sum(fw, f_batch, num_segments=G)
    f_cnt = jax.ops.segment_sum(jnp.ones((G,), x.dtype), f_batch,
                                num_segments=G)
    f_g = f_g / jnp.clip(f_cnt, 1.0)[:, None]
    fdot = f_g @ a_vec

    wia = (Wi @ a_vec).reshape(D, 1)
    Wq_s = Wq * (1.0 / math.sqrt(D))

    xw, k, v, xself, ctxd = _phase1(x, x_init, W_rel, W_self, b_gcn, Wk, Wv,
                                    wia)
    ctxs = ctxd[:, 0] + jax.nn.one_hot(x_batch, G, dtype=jnp.float32) @ fdot

    table = xw.reshape(R * N, TW)
    gidx_p = _padw(edge_type * N + src, 0)
    dsts_p = _padw(dst, N)
    zacc = jnp.zeros((NACC, TW), jnp.float32)
    accs = _sc_pass_a(table, gidx_p, dsts_p, zacc)
    agg = accs[0, :N, :D] + accs[1, :N, :D]
    deg = accs[0, :N, D] + accs[1, :N, D]

    h, q = _phase2(agg, deg, xself, Wq_s)

    dstg_p = _padw(dst, 0)
    src_p = _padw(src, 0)
    mask_p = _padw(jnp.ones((E,), jnp.float32), 0.0)
    ctxs_pad = jnp.pad(ctxs, (0, NACC - N))
    scores_p, mx = _sc_pass_b(q, k, ctxs_pad, dstg_p, src_p, mask_p)
    score = scores_p.reshape(NW, EPW)[:, :E // NW].reshape(E)

    smax = jnp.max(mx, axis=0)
    smax = jnp.where(jnp.isfinite(smax), smax, 0.0)
    smax_pad = jnp.pad(smax, (0, NACC - N))

    caccs, ex_p = _sc_pass_c(v, smax_pad, scores_p, dstg_p, dsts_p, src_p,
                             zacc)
    cacc = caccs[0, :N, :D] + caccs[1, :N, :D]
    denom = caccs[0, :N, D] + caccs[1, :N, D]
    dinv = 1.0 / jnp.clip(denom, 1e-16)

    dinv_pad = jnp.pad(dinv, (0, NACC - N))
    alpha_p = _sc_pass_d(ex_p, dstg_p, dinv_pad)
    alpha = alpha_p.reshape(NW, EPW)[:, :E // NW].reshape(E)

    z = _phase3(h, cacc, dinv, W_lin, b_lin)
    return (z, alpha)


# speedup vs baseline: 6.5846x; 1.0204x over previous
import functools
import math

import jax
import jax.numpy as jnp
from jax import lax
from jax.experimental import pallas as pl
from jax.experimental.pallas import tpu as pltpu
from jax.experimental.pallas import tpu_sc as plsc

N = 10000
E = 320000
G = 16
R = 8
D = 128

BN = 1000

NW = 32
CH = 128
EPW = 10240
NCH = EPW // CH
EPAD = NW * EPW
NACC = 10240
TW = 144

def _mesh():
    return plsc.VectorSubcoreMesh(core_axis_name="c", subcore_axis_name="s")


def _sc_params(layout_passes=True):
    kw = dict(use_tc_tiling_on_sc=False)
    if not layout_passes:
        kw["needs_layout_passes"] = False
    return pltpu.CompilerParams(**kw)


def _sc_pass_a(table, gidx_p, dsts_p, zacc):
    @functools.partial(
        pl.kernel,
        mesh=_mesh(),
        compiler_params=pltpu.CompilerParams(use_tc_tiling_on_sc=False),
        out_type=jax.ShapeDtypeStruct((2, NACC, TW), jnp.float32),
        scratch_types=[
            pltpu.VMEM((2, CH), jnp.int32),
            pltpu.VMEM((2, CH), jnp.int32),
            pltpu.VMEM((2, CH, TW), jnp.float32),
            pltpu.VMEM_SHARED((NACC, TW), jnp.float32),
            pltpu.SemaphoreType.DMA((2,)),
            pltpu.SemaphoreType.DMA((2,)),
            pltpu.SemaphoreType.DMA((2,)),
        ],
    )
    def k(table_h, gidx_h, dst_h, zacc_h, out_h, gi_v, di_v, rows_v, acc_sh,
          semi, semg, sems):
        c = lax.axis_index("c")
        s = lax.axis_index("s")
        wid = c * 16 + s
        rpt = NACC // 16
        rows0 = s * rpt
        pltpu.sync_copy(zacc_h.at[pl.ds(rows0, rpt)],
                        acc_sh.at[pl.ds(rows0, rpt)])
        plsc.subcore_barrier()
        base0 = wid * EPW
        NH = NCH // 2

        def idx_start(b, ch):
            base = base0 + ch * CH
            pltpu.async_copy(gidx_h.at[pl.ds(base, CH)], gi_v.at[b],
                             semi.at[b])
            pltpu.async_copy(dst_h.at[pl.ds(base, CH)], di_v.at[b],
                             semi.at[b])

        def idx_wait(b):
            pltpu.make_async_copy(gidx_h.at[pl.ds(base0, CH)], gi_v.at[b],
                                  semi.at[b]).wait()
            pltpu.make_async_copy(dst_h.at[pl.ds(base0, CH)], di_v.at[b],
                                  semi.at[b]).wait()

        def g_start(b):
            pltpu.async_copy(table_h.at[gi_v.at[b]], rows_v.at[b],
                             semg.at[b])

        def g_wait(b):
            pltpu.make_async_copy(table_h.at[gi_v.at[b]], rows_v.at[b],
                                  semg.at[b]).wait()

        def scat_start(b):
            pltpu.async_copy(rows_v.at[b], acc_sh.at[di_v.at[b]],
                             sems.at[b], add=True)

        def scat_wait(b):
            pltpu.make_async_copy(rows_v.at[b], acc_sh.at[di_v.at[b]],
                                  sems.at[b]).wait()

        idx_start(0, 0)
        idx_wait(0)
        g_start(0)
        idx_start(1, 1)

        @pl.loop(0, NH)
        def _(t):
            a = 2 * t
            idx_wait(1)
            g_start(1)
            g_wait(0)
            scat_start(0)
            g_wait(1)
            scat_start(1)

            @pl.when(t < NH - 1)
            def _():
                scat_wait(0)
                idx_start(0, a + 2)
                idx_wait(0)
                g_start(0)
                scat_wait(1)
                idx_start(1, a + 3)

        scat_wait(0)
        scat_wait(1)
        plsc.subcore_barrier()
        pltpu.sync_copy(acc_sh.at[pl.ds(rows0, rpt)],
                        out_h.at[c, pl.ds(rows0, rpt)])

    return k(table, gidx_p, dsts_p, zacc)


def _sc_pass_b(q, kt, ctxs_pad, dstg_p, src_p, mask_p):
    @functools.partial(
        pl.kernel,
        mesh=_mesh(),
        compiler_params=_sc_params(layout_passes=False),
        out_type=(
            jax.ShapeDtypeStruct((EPAD,), jnp.float32),
            jax.ShapeDtypeStruct((NW, N), jnp.float32),
        ),
        scratch_types=[
            pltpu.VMEM((2, CH), jnp.int32),
            pltpu.VMEM((2, CH), jnp.int32),
            pltpu.VMEM((2, CH), jnp.float32),
            pltpu.VMEM((2, CH, D), jnp.float32),
            pltpu.VMEM((2, CH, D), jnp.float32),
            pltpu.VMEM((2, CH), jnp.float32),
            pltpu.VMEM((NACC,), jnp.float32),
            pltpu.VMEM((N,), jnp.float32),
            pltpu.SemaphoreType.DMA((2,)),
            pltpu.SemaphoreType.DMA((2,)),
            pltpu.SemaphoreType.DMA((2,)),
        ],
    )
    def k(q_h, k_h, ctxs_h, dstg_h, src_h, mask_h, sco_h, mx_h,
          di_v, si_v, mk_v, qbuf, kbuf, sc_v, ctxs_v, mx_v,
          semi, semg, semo):
        c = lax.axis_index("c")
        s = lax.axis_index("s")
        wid = c * 16 + s
        pltpu.sync_copy(ctxs_h, ctxs_v)
        neg_inf = jnp.full((16,), -jnp.inf, jnp.float32)

        @pl.loop(0, N, step=16)
        def _(i):
            mx_v[pl.ds(i, 16)] = neg_inf

        base0 = wid * EPW
        NH = NCH // 2

        def idx_start(b, ch):
            base = base0 + ch * CH
            pltpu.async_copy(dstg_h.at[pl.ds(base, CH)], di_v.at[b],
                             semi.at[b])
            pltpu.async_copy(src_h.at[pl.ds(base, CH)], si_v.at[b],
                             semi.at[b])
            pltpu.async_copy(mask_h.at[pl.ds(base, CH)], mk_v.at[b],
                             semi.at[b])

        def idx_wait(b):
            pltpu.make_async_copy(dstg_h.at[pl.ds(base0, CH)], di_v.at[b],
                                  semi.at[b]).wait()
            pltpu.make_async_copy(src_h.at[pl.ds(base0, CH)], si_v.at[b],
                                  semi.at[b]).wait()
            pltpu.make_async_copy(mask_h.at[pl.ds(base0, CH)], mk_v.at[b],
                                  semi.at[b]).wait()

        def g_start(b):
            pltpu.async_copy(q_h.at[di_v.at[b]], qbuf.at[b], semg.at[b])
            pltpu.async_copy(k_h.at[si_v.at[b]], kbuf.at[b], semg.at[b])

        def g_wait(b):
            pltpu.make_async_copy(q_h.at[di_v.at[b]], qbuf.at[b],
                                  semg.at[b]).wait()
            pltpu.make_async_copy(k_h.at[si_v.at[b]], kbuf.at[b],
                                  semg.at[b]).wait()

        def out_start(b, ch):
            base = base0 + ch * CH
            pltpu.async_copy(sc_v.at[b], sco_h.at[pl.ds(base, CH)],
                             semo.at[b])

        def out_wait(b):
            pltpu.make_async_copy(sc_v.at[b], sco_h.at[pl.ds(base0, CH)],
                                  semo.at[b]).wait()

        def compute(b):
            dib = di_v.at[b]
            mkb = mk_v.at[b]
            qb = qbuf.at[b]
            kb = kbuf.at[b]
            scb = sc_v.at[b]

            @pl.loop(0, CH, step=16)
            def _(e0):
                lanes = lax.iota(jnp.int32, 16)
                acc = jnp.zeros((16,), jnp.float32)
                for e in range(16):
                    p = qb[e0 + e, pl.ds(0, 16)] * kb[e0 + e, pl.ds(0, 16)]
                    for blk in range(1, D // 16):
                        p = p + (qb[e0 + e, pl.ds(blk * 16, 16)]
                                 * kb[e0 + e, pl.ds(blk * 16, 16)])
                    acc = jnp.where(lanes == e, jnp.sum(p), acc)
                di16 = dib[pl.ds(e0, 16)]
                s16 = acc + plsc.load_gather(ctxs_v, [di16])
                s16 = jnp.where(s16 >= 0, s16, 0.2 * s16)
                s16 = jnp.where(mkb[pl.ds(e0, 16)] > 0, s16, neg_inf)
                scb[pl.ds(e0, 16)] = s16

                def mx_body(_):
                    m = plsc.load_gather(mx_v, [di16])
                    need = s16 > m
                    plsc.store_scatter(mx_v, [di16], s16, mask=need)
                    m2 = plsc.load_gather(mx_v, [di16])
                    return jnp.any(s16 > m2)

                lax.while_loop(lambda cont: cont, mx_body, jnp.bool_(True))

        idx_start(0, 0)
        idx_wait(0)
        g_start(0)
        idx_start(1, 1)

        @pl.loop(0, NH)
        def _(t):
            a = 2 * t
            idx_wait(1)
            g_start(1)
            g_wait(0)

            @pl.when(t > 0)
            def _():
                out_wait(0)

            compute(0)
            out_start(0, a)

            @pl.when(t < NH - 1)
            def _():
                idx_start(0, a + 2)
                idx_wait(0)
                g_start(0)

            g_wait(1)

            @pl.when(t > 0)
            def _():
                out_wait(1)

            compute(1)
            out_start(1, a + 1)

            @pl.when(t < NH - 1)
            def _():
                idx_start(1, a + 3)

        out_wait(0)
        out_wait(1)
        pltpu.sync_copy(mx_v, mx_h.at[wid])

    return k(q, kt, ctxs_pad, dstg_p, src_p, mask_p)


CHC = 64
NCH_C = EPW // CHC


def _sc_pass_c(vt, smax_pad, scores_p, dstg_p, dsts_p, src_p, zacc):
    @functools.partial(
        pl.kernel,
        mesh=_mesh(),
        compiler_params=_sc_params(layout_passes=False),
        out_type=(
            jax.ShapeDtypeStruct((2, NACC, TW), jnp.float32),
            jax.ShapeDtypeStruct((EPAD,), jnp.float32),
        ),
        scratch_types=[
            pltpu.VMEM((2, CHC), jnp.int32),
            pltpu.VMEM((2, CHC), jnp.int32),
            pltpu.VMEM((2, CHC), jnp.int32),
            pltpu.VMEM((2, CHC), jnp.float32),
            pltpu.VMEM((2, CHC), jnp.float32),
            pltpu.VMEM((2, CHC, TW), jnp.float32),
            pltpu.VMEM((NACC,), jnp.float32),
            pltpu.VMEM_SHARED((NACC, TW), jnp.float32),
            pltpu.SemaphoreType.DMA((2,)),
            pltpu.SemaphoreType.DMA((2,)),
            pltpu.SemaphoreType.DMA((2,)),
            pltpu.SemaphoreType.DMA((2,)),
        ],
    )
    def k(vt_h, smax_h, sco_h, dstg_h, dsts_h, src_h, zacc_h, out_h, exo_h,
          di_v, ds_v, si_v, sc_v, ex_v, vbuf, smax_v, acc_sh,
          semi, semg, sems, semo):
        c = lax.axis_index("c")
        s = lax.axis_index("s")
        wid = c * 16 + s
        rpt = NACC // 16
        rows0 = s * rpt
        pltpu.sync_copy(zacc_h.at[pl.ds(rows0, rpt)],
                        acc_sh.at[pl.ds(rows0, rpt)])
        pltpu.sync_copy(smax_h, smax_v)
        plsc.subcore_barrier()
        base0 = wid * EPW
        NH = NCH_C // 2

        def idx_start(b, ch):
            base = base0 + ch * CHC
            pltpu.async_copy(dstg_h.at[pl.ds(base, CHC)], di_v.at[b],
                             semi.at[b])
            pltpu.async_copy(dsts_h.at[pl.ds(base, CHC)], ds_v.at[b],
                             semi.at[b])
            pltpu.async_copy(src_h.at[pl.ds(base, CHC)], si_v.at[b],
                             semi.at[b])
            pltpu.async_copy(sco_h.at[pl.ds(base, CHC)], sc_v.at[b],
                             semi.at[b])

        def idx_wait(b):
            for dst in (di_v, ds_v, si_v):
                pltpu.make_async_copy(dstg_h.at[pl.ds(base0, CHC)],
                                      dst.at[b], semi.at[b]).wait()
            pltpu.make_async_copy(sco_h.at[pl.ds(base0, CHC)], sc_v.at[b],
                                  semi.at[b]).wait()

        def g_start(b):
            pltpu.async_copy(vt_h.at[si_v.at[b]], vbuf.at[b], semg.at[b])

        def g_wait(b):
            pltpu.make_async_copy(vt_h.at[si_v.at[b]], vbuf.at[b],
                                  semg.at[b]).wait()

        def scat_start(b):
            pltpu.async_copy(vbuf.at[b], acc_sh.at[ds_v.at[b]],
                             sems.at[b], add=True)

        def scat_wait(b):
            pltpu.make_async_copy(vbuf.at[b], acc_sh.at[ds_v.at[b]],
                                  sems.at[b]).wait()

        def out_start(b, ch):
            base = base0 + ch * CHC
            pltpu.async_copy(ex_v.at[b], exo_h.at[pl.ds(base, CHC)],
                             semo.at[b])

        def out_wait(b):
            pltpu.make_async_copy(ex_v.at[b], exo_h.at[pl.ds(base0, CHC)],
                                  semo.at[b]).wait()

        def compute(b):
            dib = di_v.at[b]
            scb = sc_v.at[b]
            exb = ex_v.at[b]
            vb = vbuf.at[b]

            @pl.loop(0, CHC, step=16)
            def _(e0):
                lanes = lax.iota(jnp.int32, 16)
                di16 = dib[pl.ds(e0, 16)]
                sm16 = plsc.load_gather(smax_v, [di16])
                ex16 = jnp.exp(scb[pl.ds(e0, 16)] - sm16)
                exb[pl.ds(e0, 16)] = ex16
                for e in range(16):
                    exs = jnp.sum(jnp.where(lanes == e, ex16, 0.0))
                    for blk in range(TW // 16):
                        sl = pl.ds(blk * 16, 16)
                        vb[e0 + e, sl] = vb[e0 + e, sl] * exs

        idx_start(0, 0)
        idx_wait(0)
        g_start(0)
        idx_start(1, 1)

        @pl.loop(0, NH)
        def _(t):
            a = 2 * t
            idx_wait(1)
            g_start(1)

            @pl.when(t > 0)
            def _():
                out_wait(0)

            g_wait(0)
            compute(0)
            scat_start(0)
            out_start(0, a)
            g_wait(1)

            @pl.when(t > 0)
            def _():
                out_wait(1)

            @pl.when(t < NH - 1)
            def _():
                scat_wait(0)
                idx_start(0, a + 2)
                idx_wait(0)
                g_start(0)

            compute(1)
            scat_start(1)
            out_start(1, a + 1)

            @pl.when(t < NH - 1)
            def _():
                scat_wait(1)
                idx_start(1, a + 3)

        scat_wait(0)
        scat_wait(1)
        out_wait(0)
        out_wait(1)
        plsc.subcore_barrier()
        pltpu.sync_copy(acc_sh.at[pl.ds(rows0, rpt)],
                        out_h.at[c, pl.ds(rows0, rpt)])

    return k(vt, smax_pad, scores_p, dstg_p, dsts_p, src_p, zacc)


def _sc_pass_d(ex_p, dstg_p, dinv_pad):
    @functools.partial(
        pl.kernel,
        mesh=_mesh(),
        compiler_params=_sc_params(layout_passes=False),
        out_type=jax.ShapeDtypeStruct((EPAD,), jnp.float32),
        scratch_types=[
            pltpu.VMEM((CH,), jnp.int32),
            pltpu.VMEM((CH,), jnp.float32),
            pltpu.VMEM((CH,), jnp.float32),
            pltpu.VMEM((NACC,), jnp.float32),
        ],
    )
    def k(ex_h, dstg_h, dinv_h, al_h, di_v, ex_v, al_v, dinv_v):
        c = lax.axis_index("c")
        s = lax.axis_index("s")
        wid = c * 16 + s
        pltpu.sync_copy(dinv_h, dinv_v)
        base0 = wid * EPW

        @pl.loop(0, NCH)
        def _(j):
            base = base0 + j * CH
            pltpu.sync_copy(dstg_h.at[pl.ds(base, CH)], di_v)
            pltpu.sync_copy(ex_h.at[pl.ds(base, CH)], ex_v)

            @pl.loop(0, CH, step=16)
            def _(e0):
                di16 = di_v[pl.ds(e0, 16)]
                al_v[pl.ds(e0, 16)] = (ex_v[pl.ds(e0, 16)]
                                       * plsc.load_gather(dinv_v, [di16]))

            pltpu.sync_copy(al_v, al_h.at[pl.ds(base, CH)])

    return k(ex_p, dstg_p, dinv_pad)


def _padw(a, fill):
    return jnp.pad(a.reshape(NW, E // NW), ((0, 0), (0, EPW - E // NW)),
                   constant_values=fill).reshape(-1)


def _dot(a, b):
    return lax.dot_general(a, b, (((1,), (0,)), ((), ())),
                           precision=lax.Precision.HIGHEST,
                           preferred_element_type=jnp.float32)


def _phase1_body(x_ref, xi_ref, wrel_ref, wself_ref, bgcn_ref, wk_ref, wv_ref,
                 wia_ref, xw_ref, k_ref, v_ref, xself_ref, ctxd_ref):
    xb = x_ref[...]
    ones = jnp.ones((xb.shape[0], TW - D), jnp.float32)
    for r in range(R):
        xw_ref[r] = jnp.concatenate([_dot(xb, wrel_ref[r]), ones], axis=1)
    k_ref[...] = _dot(xb, wk_ref[...])
    v_ref[...] = jnp.concatenate([_dot(xb, wv_ref[...]), ones], axis=1)
    xself_ref[...] = _dot(xb, wself_ref[...]) + bgcn_ref[...]
    ctxd_ref[...] = _dot(xi_ref[...], wia_ref[...])


def _phase1(x, x_init, W_rel, W_self, b_gcn, Wk, Wv, wia):
    grid = (N // BN,)
    full = lambda i: (0, 0)
    out_shapes = (
        jax.ShapeDtypeStruct((R, N, TW), jnp.float32),
        jax.ShapeDtypeStruct((N, D), jnp.float32),
        jax.ShapeDtypeStruct((N, TW), jnp.float32),
        jax.ShapeDtypeStruct((N, D), jnp.float32),
        jax.ShapeDtypeStruct((N, 1), jnp.float32),
    )
    return pl.pallas_call(
        _phase1_body,
        grid=grid,
        in_specs=[
            pl.BlockSpec((BN, D), lambda i: (i, 0)),
            pl.BlockSpec((BN, D), lambda i: (i, 0)),
            pl.BlockSpec((R, D, D), lambda i: (0, 0, 0)),
            pl.BlockSpec((D, D), full),
            pl.BlockSpec((1, D), full),
            pl.BlockSpec((D, D), full),
            pl.BlockSpec((D, D), full),
            pl.BlockSpec((D, 1), full),
        ],
        out_specs=(
            pl.BlockSpec((R, BN, TW), lambda i: (0, i, 0)),
            pl.BlockSpec((BN, D), lambda i: (i, 0)),
            pl.BlockSpec((BN, TW), lambda i: (i, 0)),
            pl.BlockSpec((BN, D), lambda i: (i, 0)),
            pl.BlockSpec((BN, 1), lambda i: (i, 0)),
        ),
        out_shape=out_shapes,
    )(x, x_init, W_rel, W_self, b_gcn.reshape(1, D), Wk, Wv, wia)


def _phase2_body(agg_ref, deg_ref, xself_ref, wq_ref, h_ref, q_ref):
    degc = jnp.maximum(deg_ref[...], 1.0)
    h = agg_ref[...] / degc + xself_ref[...]
    h_ref[...] = h
    q_ref[...] = _dot(h, wq_ref[...])


def _phase2(agg, deg, xself, Wq_s):
    grid = (N // BN,)
    return pl.pallas_call(
        _phase2_body,
        grid=grid,
        in_specs=[
            pl.BlockSpec((BN, D), lambda i: (i, 0)),
            pl.BlockSpec((BN, 1), lambda i: (i, 0)),
            pl.BlockSpec((BN, D), lambda i: (i, 0)),
            pl.BlockSpec((D, D), lambda i: (0, 0)),
        ],
        out_specs=(
            pl.BlockSpec((BN, D), lambda i: (i, 0)),
            pl.BlockSpec((BN, D), lambda i: (i, 0)),
        ),
        out_shape=(
            jax.ShapeDtypeStruct((N, D), jnp.float32),
            jax.ShapeDtypeStruct((N, D), jnp.float32),
        ),
    )(agg, deg.reshape(N, 1), xself, Wq_s)


def _phase3_body(h_ref, cacc_ref, dinv_ref, wlin_ref, blin_ref, z_ref):
    z = h_ref[...] + cacc_ref[...] * dinv_ref[...]
    z_ref[...] = _dot(z, wlin_ref[...]) + blin_ref[...]


def _phase3(h, cacc, dinv, W_lin, b_lin):
    grid = (N // BN,)
    return pl.pallas_call(
        _phase3_body,
        grid=grid,
        in_specs=[
            pl.BlockSpec((BN, D), lambda i: (i, 0)),
            pl.BlockSpec((BN, D), lambda i: (i, 0)),
            pl.BlockSpec((BN, 1), lambda i: (i, 0)),
            pl.BlockSpec((D, D), lambda i: (0, 0)),
            pl.BlockSpec((1, D), lambda i: (0, 0)),
        ],
        out_specs=pl.BlockSpec((BN, D), lambda i: (i, 0)),
        out_shape=jax.ShapeDtypeStruct((N, D), jnp.float32),
    )(h, cacc, dinv.reshape(N, 1), W_lin, b_lin.reshape(1, D))


def kernel(f, x, x_init, edge_index, edge_type, f_batch, x_batch, W_rel,
           W_self, b_gcn, Wq, Wk, Wv, Wf, Wi, a_vec, W_lin, b_lin):
    src = edge_index[0]
    dst = edge_index[1]

    fw = f @ Wf
    f_g = jax.ops.segment_sum(fw, f_batch, num_segments=G)
    f_cnt = jax.ops.segment_sum(jnp.ones((G,), x.dtype), f_batch,
                                num_segments=G)
    f_g = f_g / jnp.clip(f_cnt, 1.0)[:, None]
    fdot = f_g @ a_vec

    wia = (Wi @ a_vec).reshape(D, 1)
    Wq_s = Wq * (1.0 / math.sqrt(D))

    xw, k, v, xself, ctxd = _phase1(x, x_init, W_rel, W_self, b_gcn, Wk, Wv,
                                    wia)
    ctxs = ctxd[:, 0] + jax.nn.one_hot(x_batch, G, dtype=jnp.float32) @ fdot

    table = xw.reshape(R * N, TW)
    gidx_p = _padw(edge_type * N + src, 0)
    dsts_p = _padw(dst, N)
    zacc = jnp.zeros((NACC, TW), jnp.float32)
    accs = _sc_pass_a(table, gidx_p, dsts_p, zacc)
    agg = accs[0, :N, :D] + accs[1, :N, :D]
    deg = accs[0, :N, D] + accs[1, :N, D]

    h, q = _phase2(agg, deg, xself, Wq_s)

    dstg_p = _padw(dst, 0)
    src_p = _padw(src, 0)
    mask_p = _padw(jnp.ones((E,), jnp.float32), 0.0)
    ctxs_pad = jnp.pad(ctxs, (0, NACC - N))
    scores_p, mx = _sc_pass_b(q, k, ctxs_pad, dstg_p, src_p, mask_p)
    score = scores_p.reshape(NW, EPW)[:, :E // NW].reshape(E)

    smax = jnp.max(mx, axis=0)
    smax = jnp.where(jnp.isfinite(smax), smax, 0.0)
    smax_pad = jnp.pad(smax, (0, NACC - N))

    caccs, ex_p = _sc_pass_c(v, smax_pad, scores_p, dstg_p, dsts_p, src_p,
                             zacc)
    cacc = caccs[0, :N, :D] + caccs[1, :N, :D]
    denom = caccs[0, :N, D] + caccs[1, :N, D]
    dinv = 1.0 / jnp.clip(denom, 1e-16)

    dinv_pad = jnp.pad(dinv, (0, NACC - N))
    alpha_p = _sc_pass_d(ex_p, dstg_p, dinv_pad)
    alpha = alpha_p.reshape(NW, EPW)[:, :E // NW].reshape(E)

    z = _phase3(h, cacc, dinv, W_lin, b_lin)
    return (z, alpha)


# speedup vs baseline: 7.0795x; 1.0752x over previous
import functools
import math

import jax
import jax.numpy as jnp
from jax import lax
from jax.experimental import pallas as pl
from jax.experimental.pallas import tpu as pltpu
from jax.experimental.pallas import tpu_sc as plsc

N = 10000
E = 320000
G = 16
R = 8
D = 128

BN = 1000

NW = 32
CH = 128
EPW = 10240
NCH = EPW // CH
EPAD = NW * EPW
NACC = 10240
TW = D
NACCR = 10016

def _mesh():
    return plsc.VectorSubcoreMesh(core_axis_name="c", subcore_axis_name="s")


def _sc_params(layout_passes=True):
    kw = dict(use_tc_tiling_on_sc=False)
    if not layout_passes:
        kw["needs_layout_passes"] = False
    return pltpu.CompilerParams(**kw)


def _sc_pass_a(table, gidx_p, dsts_p, zacc):
    @functools.partial(
        pl.kernel,
        mesh=_mesh(),
        compiler_params=_sc_params(layout_passes=False),
        out_type=(jax.ShapeDtypeStruct((2, NACCR, TW), jnp.float32),
                  jax.ShapeDtypeStruct((NW, N), jnp.float32)),
        scratch_types=[
            pltpu.VMEM((2, CH), jnp.int32),
            pltpu.VMEM((2, CH), jnp.int32),
            pltpu.VMEM((2, CH, TW), jnp.float32),
            pltpu.VMEM((NACCR,), jnp.float32),
            pltpu.VMEM_SHARED((NACCR, TW), jnp.float32),
            pltpu.SemaphoreType.DMA((2,)),
            pltpu.SemaphoreType.DMA((2,)),
            pltpu.SemaphoreType.DMA((2,)),
        ],
    )
    def k(table_h, gidx_h, dst_h, zacc_h, out_h, degs_h, gi_v, di_v, rows_v,
          deg_v, acc_sh, semi, semg, sems):
        c = lax.axis_index("c")
        s = lax.axis_index("s")
        wid = c * 16 + s
        rpt = NACCR // 16
        rows0 = s * rpt
        pltpu.sync_copy(zacc_h.at[pl.ds(rows0, rpt)],
                        acc_sh.at[pl.ds(rows0, rpt)])
        zero16 = jnp.zeros((16,), jnp.float32)
        one16 = jnp.ones((16,), jnp.float32)

        @pl.loop(0, NACCR, step=16)
        def _(i):
            deg_v[pl.ds(i, 16)] = zero16

        plsc.subcore_barrier()
        base0 = wid * EPW
        NH = NCH // 2

        def deg_acc(b):
            dib = di_v.at[b]

            @pl.loop(0, CH, step=16)
            def _(e0):
                plsc.addupdate_scatter(deg_v, [dib[pl.ds(e0, 16)]], one16)

        def idx_start(b, ch):
            base = base0 + ch * CH
            pltpu.async_copy(gidx_h.at[pl.ds(base, CH)], gi_v.at[b],
                             semi.at[b])
            pltpu.async_copy(dst_h.at[pl.ds(base, CH)], di_v.at[b],
                             semi.at[b])

        def idx_wait(b):
            pltpu.make_async_copy(gidx_h.at[pl.ds(base0, CH)], gi_v.at[b],
                                  semi.at[b]).wait()
            pltpu.make_async_copy(dst_h.at[pl.ds(base0, CH)], di_v.at[b],
                                  semi.at[b]).wait()

        def g_start(b):
            pltpu.async_copy(table_h.at[gi_v.at[b]], rows_v.at[b],
                             semg.at[b])

        def g_wait(b):
            pltpu.make_async_copy(table_h.at[gi_v.at[b]], rows_v.at[b],
                                  semg.at[b]).wait()

        def scat_start(b):
            pltpu.async_copy(rows_v.at[b], acc_sh.at[di_v.at[b]],
                             sems.at[b], add=True)

        def scat_wait(b):
            pltpu.make_async_copy(rows_v.at[b], acc_sh.at[di_v.at[b]],
                                  sems.at[b]).wait()

        idx_start(0, 0)
        idx_wait(0)
        g_start(0)
        idx_start(1, 1)

        @pl.loop(0, NH)
        def _(t):
            a = 2 * t
            idx_wait(1)
            g_start(1)
            g_wait(0)
            scat_start(0)
            deg_acc(0)
            g_wait(1)
            scat_start(1)
            deg_acc(1)

            @pl.when(t < NH - 1)
            def _():
                scat_wait(0)
                idx_start(0, a + 2)
                idx_wait(0)
                g_start(0)
                scat_wait(1)
                idx_start(1, a + 3)

        scat_wait(0)
        scat_wait(1)
        pltpu.sync_copy(deg_v.at[pl.ds(0, N)], degs_h.at[wid])
        plsc.subcore_barrier()
        pltpu.sync_copy(acc_sh.at[pl.ds(rows0, rpt)],
                        out_h.at[c, pl.ds(rows0, rpt)])

    return k(table, gidx_p, dsts_p, zacc)


def _sc_pass_b(q, kt, ctxs_pad, dstg_p, src_p, mask_p):
    @functools.partial(
        pl.kernel,
        mesh=_mesh(),
        compiler_params=_sc_params(layout_passes=False),
        out_type=(
            jax.ShapeDtypeStruct((EPAD,), jnp.float32),
            jax.ShapeDtypeStruct((NW, N), jnp.float32),
        ),
        scratch_types=[
            pltpu.VMEM((2, CH), jnp.int32),
            pltpu.VMEM((2, CH), jnp.int32),
            pltpu.VMEM((2, CH), jnp.float32),
            pltpu.VMEM((2, CH, D), jnp.float32),
            pltpu.VMEM((2, CH, D), jnp.float32),
            pltpu.VMEM((2, CH), jnp.float32),
            pltpu.VMEM((NACC,), jnp.float32),
            pltpu.VMEM((N,), jnp.float32),
            pltpu.SemaphoreType.DMA((2,)),
            pltpu.SemaphoreType.DMA((2,)),
            pltpu.SemaphoreType.DMA((2,)),
        ],
    )
    def k(q_h, k_h, ctxs_h, dstg_h, src_h, mask_h, sco_h, mx_h,
          di_v, si_v, mk_v, qbuf, kbuf, sc_v, ctxs_v, mx_v,
          semi, semg, semo):
        c = lax.axis_index("c")
        s = lax.axis_index("s")
        wid = c * 16 + s
        pltpu.sync_copy(ctxs_h, ctxs_v)
        neg_inf = jnp.full((16,), -jnp.inf, jnp.float32)

        @pl.loop(0, N, step=16)
        def _(i):
            mx_v[pl.ds(i, 16)] = neg_inf

        base0 = wid * EPW
        NH = NCH // 2

        def idx_start(b, ch):
            base = base0 + ch * CH
            pltpu.async_copy(dstg_h.at[pl.ds(base, CH)], di_v.at[b],
                             semi.at[b])
            pltpu.async_copy(src_h.at[pl.ds(base, CH)], si_v.at[b],
                             semi.at[b])
            pltpu.async_copy(mask_h.at[pl.ds(base, CH)], mk_v.at[b],
                             semi.at[b])

        def idx_wait(b):
            pltpu.make_async_copy(dstg_h.at[pl.ds(base0, CH)], di_v.at[b],
                                  semi.at[b]).wait()
            pltpu.make_async_copy(src_h.at[pl.ds(base0, CH)], si_v.at[b],
                                  semi.at[b]).wait()
            pltpu.make_async_copy(mask_h.at[pl.ds(base0, CH)], mk_v.at[b],
                                  semi.at[b]).wait()

        def g_start(b):
            pltpu.async_copy(q_h.at[di_v.at[b]], qbuf.at[b], semg.at[b])
            pltpu.async_copy(k_h.at[si_v.at[b]], kbuf.at[b], semg.at[b])

        def g_wait(b):
            pltpu.make_async_copy(q_h.at[di_v.at[b]], qbuf.at[b],
                                  semg.at[b]).wait()
            pltpu.make_async_copy(k_h.at[si_v.at[b]], kbuf.at[b],
                                  semg.at[b]).wait()

        def out_start(b, ch):
            base = base0 + ch * CH
            pltpu.async_copy(sc_v.at[b], sco_h.at[pl.ds(base, CH)],
                             semo.at[b])

        def out_wait(b):
            pltpu.make_async_copy(sc_v.at[b], sco_h.at[pl.ds(base0, CH)],
                                  semo.at[b]).wait()

        def compute(b):
            dib = di_v.at[b]
            mkb = mk_v.at[b]
            qb = qbuf.at[b]
            kb = kbuf.at[b]
            scb = sc_v.at[b]

            @pl.loop(0, CH, step=16)
            def _(e0):
                lanes = lax.iota(jnp.int32, 16)
                acc = jnp.zeros((16,), jnp.float32)
                for e in range(16):
                    p = qb[e0 + e, pl.ds(0, 16)] * kb[e0 + e, pl.ds(0, 16)]
                    for blk in range(1, D // 16):
                        p = p + (qb[e0 + e, pl.ds(blk * 16, 16)]
                                 * kb[e0 + e, pl.ds(blk * 16, 16)])
                    acc = jnp.where(lanes == e, jnp.sum(p), acc)
                di16 = dib[pl.ds(e0, 16)]
                s16 = acc + plsc.load_gather(ctxs_v, [di16])
                s16 = jnp.where(s16 >= 0, s16, 0.2 * s16)
                s16 = jnp.where(mkb[pl.ds(e0, 16)] > 0, s16, neg_inf)
                scb[pl.ds(e0, 16)] = s16

                def mx_body(_):
                    m = plsc.load_gather(mx_v, [di16])
                    need = s16 > m
                    plsc.store_scatter(mx_v, [di16], s16, mask=need)
                    m2 = plsc.load_gather(mx_v, [di16])
                    return jnp.any(s16 > m2)

                lax.while_loop(lambda cont: cont, mx_body, jnp.bool_(True))

        idx_start(0, 0)
        idx_wait(0)
        g_start(0)
        idx_start(1, 1)

        @pl.loop(0, NH)
        def _(t):
            a = 2 * t
            idx_wait(1)
            g_start(1)
            g_wait(0)

            @pl.when(t > 0)
            def _():
                out_wait(0)

            compute(0)
            out_start(0, a)

            @pl.when(t < NH - 1)
            def _():
                idx_start(0, a + 2)
                idx_wait(0)
                g_start(0)

            g_wait(1)

            @pl.when(t > 0)
            def _():
                out_wait(1)

            compute(1)
            out_start(1, a + 1)

            @pl.when(t < NH - 1)
            def _():
                idx_start(1, a + 3)

        out_wait(0)
        out_wait(1)
        pltpu.sync_copy(mx_v, mx_h.at[wid])

    return k(q, kt, ctxs_pad, dstg_p, src_p, mask_p)


CHC = 64
NCH_C = EPW // CHC


def _sc_pass_c(vt, smax_pad, scores_p, dstg_p, dsts_p, src_p, zacc):
    @functools.partial(
        pl.kernel,
        mesh=_mesh(),
        compiler_params=_sc_params(layout_passes=False),
        out_type=(
            jax.ShapeDtypeStruct((2, NACCR, TW), jnp.float32),
            jax.ShapeDtypeStruct((EPAD,), jnp.float32),
            jax.ShapeDtypeStruct((NW, N), jnp.float32),
        ),
        scratch_types=[
            pltpu.VMEM((2, CHC), jnp.int32),
            pltpu.VMEM((2, CHC), jnp.int32),
            pltpu.VMEM((2, CHC), jnp.int32),
            pltpu.VMEM((2, CHC), jnp.float32),
            pltpu.VMEM((2, CHC), jnp.float32),
            pltpu.VMEM((2, CHC, TW), jnp.float32),
            pltpu.VMEM((NACC,), jnp.float32),
            pltpu.VMEM((NACCR,), jnp.float32),
            pltpu.VMEM_SHARED((NACCR, TW), jnp.float32),
            pltpu.SemaphoreType.DMA((2,)),
            pltpu.SemaphoreType.DMA((2,)),
            pltpu.SemaphoreType.DMA((2,)),
            pltpu.SemaphoreType.DMA((2,)),
        ],
    )
    def k(vt_h, smax_h, sco_h, dstg_h, dsts_h, src_h, zacc_h, out_h, exo_h,
          dens_h, di_v, ds_v, si_v, sc_v, ex_v, vbuf, smax_v, den_v, acc_sh,
          semi, semg, sems, semo):
        c = lax.axis_index("c")
        s = lax.axis_index("s")
        wid = c * 16 + s
        rpt = NACCR // 16
        rows0 = s * rpt
        pltpu.sync_copy(zacc_h.at[pl.ds(rows0, rpt)],
                        acc_sh.at[pl.ds(rows0, rpt)])
        pltpu.sync_copy(smax_h, smax_v)

        @pl.loop(0, NACCR, step=16)
        def _(i):
            den_v[pl.ds(i, 16)] = jnp.zeros((16,), jnp.float32)

        plsc.subcore_barrier()
        base0 = wid * EPW
        NH = NCH_C // 2

        def idx_start(b, ch):
            base = base0 + ch * CHC
            pltpu.async_copy(dstg_h.at[pl.ds(base, CHC)], di_v.at[b],
                             semi.at[b])
            pltpu.async_copy(dsts_h.at[pl.ds(base, CHC)], ds_v.at[b],
                             semi.at[b])
            pltpu.async_copy(src_h.at[pl.ds(base, CHC)], si_v.at[b],
                             semi.at[b])
            pltpu.async_copy(sco_h.at[pl.ds(base, CHC)], sc_v.at[b],
                             semi.at[b])

        def idx_wait(b):
            for dst in (di_v, ds_v, si_v):
                pltpu.make_async_copy(dstg_h.at[pl.ds(base0, CHC)],
                                      dst.at[b], semi.at[b]).wait()
            pltpu.make_async_copy(sco_h.at[pl.ds(base0, CHC)], sc_v.at[b],
                                  semi.at[b]).wait()

        def g_start(b):
            pltpu.async_copy(vt_h.at[si_v.at[b]], vbuf.at[b], semg.at[b])

        def g_wait(b):
            pltpu.make_async_copy(vt_h.at[si_v.at[b]], vbuf.at[b],
                                  semg.at[b]).wait()

        def scat_start(b):
            pltpu.async_copy(vbuf.at[b], acc_sh.at[ds_v.at[b]],
                             sems.at[b], add=True)

        def scat_wait(b):
            pltpu.make_async_copy(vbuf.at[b], acc_sh.at[ds_v.at[b]],
                                  sems.at[b]).wait()

        def out_start(b, ch):
            base = base0 + ch * CHC
            pltpu.async_copy(ex_v.at[b], exo_h.at[pl.ds(base, CHC)],
                             semo.at[b])

        def out_wait(b):
            pltpu.make_async_copy(ex_v.at[b], exo_h.at[pl.ds(base0, CHC)],
                                  semo.at[b]).wait()

        def compute(b):
            dib = di_v.at[b]
            scb = sc_v.at[b]
            exb = ex_v.at[b]
            vb = vbuf.at[b]

            @pl.loop(0, CHC, step=16)
            def _(e0):
                lanes = lax.iota(jnp.int32, 16)
                di16 = dib[pl.ds(e0, 16)]
                sm16 = plsc.load_gather(smax_v, [di16])
                ex16 = jnp.exp(scb[pl.ds(e0, 16)] - sm16)
                exb[pl.ds(e0, 16)] = ex16
                plsc.addupdate_scatter(den_v, [di16], ex16)
                for e in range(16):
                    exs = jnp.sum(jnp.where(lanes == e, ex16, 0.0))
                    for blk in range(TW // 16):
                        sl = pl.ds(blk * 16, 16)
                        vb[e0 + e, sl] = vb[e0 + e, sl] * exs

        idx_start(0, 0)
        idx_wait(0)
        g_start(0)
        idx_start(1, 1)

        @pl.loop(0, NH)
        def _(t):
            a = 2 * t
            idx_wait(1)
            g_start(1)

            @pl.when(t > 0)
            def _():
                out_wait(0)

            g_wait(0)
            compute(0)
            scat_start(0)
            out_start(0, a)
            g_wait(1)

            @pl.when(t > 0)
            def _():
                out_wait(1)

            @pl.when(t < NH - 1)
            def _():
                scat_wait(0)
                idx_start(0, a + 2)
                idx_wait(0)
                g_start(0)

            compute(1)
            scat_start(1)
            out_start(1, a + 1)

            @pl.when(t < NH - 1)
            def _():
                scat_wait(1)
                idx_start(1, a + 3)

        scat_wait(0)
        scat_wait(1)
        out_wait(0)
        out_wait(1)
        pltpu.sync_copy(den_v.at[pl.ds(0, N)], dens_h.at[wid])
        plsc.subcore_barrier()
        pltpu.sync_copy(acc_sh.at[pl.ds(rows0, rpt)],
                        out_h.at[c, pl.ds(rows0, rpt)])

    return k(vt, smax_pad, scores_p, dstg_p, dsts_p, src_p, zacc)


def _sc_pass_d(ex_p, dstg_p, dinv_pad):
    @functools.partial(
        pl.kernel,
        mesh=_mesh(),
        compiler_params=_sc_params(layout_passes=False),
        out_type=jax.ShapeDtypeStruct((EPAD,), jnp.float32),
        scratch_types=[
            pltpu.VMEM((CH,), jnp.int32),
            pltpu.VMEM((CH,), jnp.float32),
            pltpu.VMEM((CH,), jnp.float32),
            pltpu.VMEM((NACC,), jnp.float32),
        ],
    )
    def k(ex_h, dstg_h, dinv_h, al_h, di_v, ex_v, al_v, dinv_v):
        c = lax.axis_index("c")
        s = lax.axis_index("s")
        wid = c * 16 + s
        pltpu.sync_copy(dinv_h, dinv_v)
        base0 = wid * EPW

        @pl.loop(0, NCH)
        def _(j):
            base = base0 + j * CH
            pltpu.sync_copy(dstg_h.at[pl.ds(base, CH)], di_v)
            pltpu.sync_copy(ex_h.at[pl.ds(base, CH)], ex_v)

            @pl.loop(0, CH, step=16)
            def _(e0):
                di16 = di_v[pl.ds(e0, 16)]
                al_v[pl.ds(e0, 16)] = (ex_v[pl.ds(e0, 16)]
                                       * plsc.load_gather(dinv_v, [di16]))

            pltpu.sync_copy(al_v, al_h.at[pl.ds(base, CH)])

    return k(ex_p, dstg_p, dinv_pad)


def _padw(a, fill):
    return jnp.pad(a.reshape(NW, E // NW), ((0, 0), (0, EPW - E // NW)),
                   constant_values=fill).reshape(-1)


def _dot(a, b):
    return lax.dot_general(a, b, (((1,), (0,)), ((), ())),
                           precision=lax.Precision.HIGHEST,
                           preferred_element_type=jnp.float32)


def _phase1_body(x_ref, xi_ref, wrel_ref, wself_ref, bgcn_ref, wk_ref, wv_ref,
                 wia_ref, xw_ref, k_ref, v_ref, xself_ref, ctxd_ref):
    xb = x_ref[...]
    for r in range(R):
        xw_ref[r] = _dot(xb, wrel_ref[r])
    k_ref[...] = _dot(xb, wk_ref[...])
    v_ref[...] = _dot(xb, wv_ref[...])
    xself_ref[...] = _dot(xb, wself_ref[...]) + bgcn_ref[...]
    ctxd_ref[...] = _dot(xi_ref[...], wia_ref[...])


def _phase1(x, x_init, W_rel, W_self, b_gcn, Wk, Wv, wia):
    grid = (N // BN,)
    full = lambda i: (0, 0)
    out_shapes = (
        jax.ShapeDtypeStruct((R, N, D), jnp.float32),
        jax.ShapeDtypeStruct((N, D), jnp.float32),
        jax.ShapeDtypeStruct((N, D), jnp.float32),
        jax.ShapeDtypeStruct((N, D), jnp.float32),
        jax.ShapeDtypeStruct((N, 1), jnp.float32),
    )
    return pl.pallas_call(
        _phase1_body,
        grid=grid,
        in_specs=[
            pl.BlockSpec((BN, D), lambda i: (i, 0)),
            pl.BlockSpec((BN, D), lambda i: (i, 0)),
            pl.BlockSpec((R, D, D), lambda i: (0, 0, 0)),
            pl.BlockSpec((D, D), full),
            pl.BlockSpec((1, D), full),
            pl.BlockSpec((D, D), full),
            pl.BlockSpec((D, D), full),
            pl.BlockSpec((D, 1), full),
        ],
        out_specs=(
            pl.BlockSpec((R, BN, D), lambda i: (0, i, 0)),
            pl.BlockSpec((BN, D), lambda i: (i, 0)),
            pl.BlockSpec((BN, D), lambda i: (i, 0)),
            pl.BlockSpec((BN, D), lambda i: (i, 0)),
            pl.BlockSpec((BN, 1), lambda i: (i, 0)),
        ),
        out_shape=out_shapes,
    )(x, x_init, W_rel, W_self, b_gcn.reshape(1, D), Wk, Wv, wia)


def _phase2_body(agg_ref, deg_ref, xself_ref, wq_ref, h_ref, q_ref):
    degc = jnp.maximum(deg_ref[...], 1.0)
    h = agg_ref[...] / degc + xself_ref[...]
    h_ref[...] = h
    q_ref[...] = _dot(h, wq_ref[...])


def _phase2(agg, deg, xself, Wq_s):
    grid = (N // BN,)
    return pl.pallas_call(
        _phase2_body,
        grid=grid,
        in_specs=[
            pl.BlockSpec((BN, D), lambda i: (i, 0)),
            pl.BlockSpec((BN, 1), lambda i: (i, 0)),
            pl.BlockSpec((BN, D), lambda i: (i, 0)),
            pl.BlockSpec((D, D), lambda i: (0, 0)),
        ],
        out_specs=(
            pl.BlockSpec((BN, D), lambda i: (i, 0)),
            pl.BlockSpec((BN, D), lambda i: (i, 0)),
        ),
        out_shape=(
            jax.ShapeDtypeStruct((N, D), jnp.float32),
            jax.ShapeDtypeStruct((N, D), jnp.float32),
        ),
    )(agg, deg.reshape(N, 1), xself, Wq_s)


def _phase3_body(h_ref, cacc_ref, dinv_ref, wlin_ref, blin_ref, z_ref):
    z = h_ref[...] + cacc_ref[...] * dinv_ref[...]
    z_ref[...] = _dot(z, wlin_ref[...]) + blin_ref[...]


def _phase3(h, cacc, dinv, W_lin, b_lin):
    grid = (N // BN,)
    return pl.pallas_call(
        _phase3_body,
        grid=grid,
        in_specs=[
            pl.BlockSpec((BN, D), lambda i: (i, 0)),
            pl.BlockSpec((BN, D), lambda i: (i, 0)),
            pl.BlockSpec((BN, 1), lambda i: (i, 0)),
            pl.BlockSpec((D, D), lambda i: (0, 0)),
            pl.BlockSpec((1, D), lambda i: (0, 0)),
        ],
        out_specs=pl.BlockSpec((BN, D), lambda i: (i, 0)),
        out_shape=jax.ShapeDtypeStruct((N, D), jnp.float32),
    )(h, cacc, dinv.reshape(N, 1), W_lin, b_lin.reshape(1, D))


def kernel(f, x, x_init, edge_index, edge_type, f_batch, x_batch, W_rel,
           W_self, b_gcn, Wq, Wk, Wv, Wf, Wi, a_vec, W_lin, b_lin):
    src = edge_index[0]
    dst = edge_index[1]

    fw = f @ Wf
    f_g = jax.ops.segment_sum(fw, f_batch, num_segments=G)
    f_cnt = jax.ops.segment_sum(jnp.ones((G,), x.dtype), f_batch,
                                num_segments=G)
    f_g = f_g / jnp.clip(f_cnt, 1.0)[:, None]
    fdot = f_g @ a_vec

    wia = (Wi @ a_vec).reshape(D, 1)
    Wq_s = Wq * (1.0 / math.sqrt(D))

    xw, k, v, xself, ctxd = _phase1(x, x_init, W_rel, W_self, b_gcn, Wk, Wv,
                                    wia)
    ctxs = ctxd[:, 0] + jax.nn.one_hot(x_batch, G, dtype=jnp.float32) @ fdot

    table = xw.reshape(R * N, TW)
    gidx_p = _padw(edge_type * N + src, 0)
    dsts_p = _padw(dst, N)
    zacc = jnp.zeros((NACCR, TW), jnp.float32)
    accs, degs = _sc_pass_a(table, gidx_p, dsts_p, zacc)
    agg = accs[0, :N] + accs[1, :N]
    deg = degs.sum(axis=0)

    h, q = _phase2(agg, deg, xself, Wq_s)

    dstg_p = _padw(dst, 0)
    src_p = _padw(src, 0)
    mask_p = _padw(jnp.ones((E,), jnp.float32), 0.0)
    ctxs_pad = jnp.pad(ctxs, (0, NACC - N))
    scores_p, mx = _sc_pass_b(q, k, ctxs_pad, dstg_p, src_p, mask_p)
    score = scores_p.reshape(NW, EPW)[:, :E // NW].reshape(E)

    smax = jnp.max(mx, axis=0)
    smax = jnp.where(jnp.isfinite(smax), smax, 0.0)
    smax_pad = jnp.pad(smax, (0, NACC - N))

    caccs, ex_p, dens = _sc_pass_c(v, smax_pad, scores_p, dstg_p, dsts_p,
                                   src_p, zacc)
    cacc = caccs[0, :N] + caccs[1, :N]
    denom = dens.sum(axis=0)
    dinv = 1.0 / jnp.clip(denom, 1e-16)

    dinv_pad = jnp.pad(dinv, (0, NACC - N))
    alpha_p = _sc_pass_d(ex_p, dstg_p, dinv_pad)
    alpha = alpha_p.reshape(NW, EPW)[:, :E // NW].reshape(E)

    z = _phase3(h, cacc, dinv, W_lin, b_lin)
    return (z, alpha)
